# Initial kernel scaffold; baseline (speedup 1.0000x reference)
import jax
import jax.numpy as jnp
from jax.experimental import pallas as pl


def kernel(user_emb, entity_emb, edge_index, edge_type, mat_row, mat_col, mat_val, weight, gate1_w0, gate2_w0, gate1_w1, gate2_w1):
    raise NotImplementedError("write your pallas kernel here")


# speedup vs baseline: 1.0518x; 1.0518x over previous
import jax
import jax.numpy as jnp
from jax.experimental import pallas as pl

N_USERS = 30000
N_ITEMS = 20000
N_ENTITIES = 50000
N_RELATIONS = 16
DIM = 64
N_HOPS = 2


def _norm_body(x_ref, out_ref):
    x = x_ref[...]
    n = jnp.sqrt(jnp.sum(x * x, axis=1, keepdims=True))
    out_ref[...] = x / jnp.maximum(n, 1e-12)


def _normalize(x):
    n = x.shape[0]
    blk = 1000
    assert n % blk == 0
    return pl.pallas_call(
        _norm_body,
        grid=(n // blk,),
        in_specs=[pl.BlockSpec((blk, DIM), lambda i: (i, 0))],
        out_specs=pl.BlockSpec((blk, DIM), lambda i: (i, 0)),
        out_shape=jax.ShapeDtypeStruct((n, DIM), x.dtype),
    )(x)


def _scatter_mean(src, index, dim_size):
    s = jax.ops.segment_sum(src, index, num_segments=dim_size)
    c = jax.ops.segment_sum(jnp.ones((src.shape[0], 1), src.dtype), index, num_segments=dim_size)
    return s / jnp.clip(c, 1.0, None)


def kernel(user_emb, entity_emb, edge_index, edge_type, mat_row, mat_col, mat_val,
           weight, gate1_w0, gate2_w0, gate1_w1, gate2_w1):
    gate_ws = [(gate1_w0, gate2_w0), (gate1_w1, gate2_w1)]
    head = edge_index[0]
    tail = edge_index[1]
    entity_res = entity_emb
    user_res = user_emb
    e_emb = entity_emb
    u_emb = user_emb
    for i in range(N_HOPS):
        edge_rel = weight[edge_type]
        neigh = e_emb[tail] * edge_rel
        entity_agg = _scatter_mean(neigh, head, N_ENTITIES)
        item_kg_agg = entity_agg[:N_ITEMS]
        att_kg_agg = entity_agg[N_ITEMS:]
        item_neigh = u_emb[mat_row] * weight[0]
        i_u_agg = _scatter_mean(item_neigh, mat_col, N_ITEMS)
        g1, g2 = gate_ws[i]
        gi = jax.nn.sigmoid(item_kg_agg @ g1.T + i_u_agg @ g2.T)
        item_fusion = gi * item_kg_agg + (1.0 - gi) * i_u_agg
        user_agg = jax.ops.segment_sum(item_fusion[mat_col], mat_row, num_segments=N_USERS)
        e_emb = _normalize(jnp.concatenate([item_fusion, att_kg_agg], axis=0))
        u_emb = _normalize(user_agg)
        entity_res = entity_res + e_emb
        user_res = user_res + u_emb
    return (entity_res, user_res)


# speedup vs baseline: 2.6715x; 2.5399x over previous
import functools

import jax
import jax.numpy as jnp
from jax import lax
from jax.experimental import pallas as pl
from jax.experimental.pallas import tpu as pltpu
from jax.experimental.pallas import tpu_sc as plsc

N_USERS = 30000
N_ITEMS = 20000
N_ENTITIES = 50000
N_RELATIONS = 16
DIM = 64
HALF = 32
N_HOPS = 2
N_EDGES = 800000
N_INTER = 500000

NC = 2
NS = 16
NW = NC * NS
L = 16
CHUNK = 128

E_CH = 196
I_CH = 123
B_E = 14
B_I = 41

ENT_ACC = 51200
ITEM_ACC = 20480
USER_ACC = 30720

_mesh = plsc.VectorSubcoreMesh(
    core_axis_name="c", subcore_axis_name="s", num_cores=NC, num_subcores=NS)
_sc_params = pltpu.CompilerParams(use_tc_tiling_on_sc=False)


def _zero_rows(buf, nrows):
    @pl.loop(0, nrows, unroll=8)
    def _(r):
        z = jnp.zeros((L,), jnp.float32)
        buf[r, pl.ds(0, L)] = z
        buf[r, pl.ds(L, L)] = z


def _zero_flat(buf, n):
    @pl.loop(0, n // L, unroll=8)
    def _(k):
        buf[pl.ds(k * L, L)] = jnp.zeros((L,), jnp.float32)


@functools.partial(
    pl.kernel,
    out_type=(jax.ShapeDtypeStruct((ENT_ACC,), jnp.float32),
              jax.ShapeDtypeStruct((ITEM_ACC,), jnp.float32)),
    mesh=_mesh,
    compiler_params=_sc_params,
    scratch_types=(
        pltpu.VMEM_SHARED((ENT_ACC,), jnp.float32),
        pltpu.VMEM((B_E, CHUNK), jnp.int32),
        pltpu.VMEM((CHUNK,), jnp.float32),
        pltpu.VMEM((3200,), jnp.float32),
    ),
)
def _sc_counts(head2, cols2, cnt_e, cnt_i, acc, idx_v, ones_v, stage_v):
    c = lax.axis_index("c")
    s = lax.axis_index("s")
    @pl.loop(0, CHUNK // L, unroll=8)
    def _(k):
        ones_v[pl.ds(k * L, L)] = jnp.ones((L,), jnp.float32)
    _zero_flat(stage_v, 3200)

    @pl.when(c == 0)
    def _():
        pltpu.sync_copy(stage_v, acc.at[pl.ds(s * 3200, 3200)])
    @pl.when(c == 1)
    def _():
        pltpu.sync_copy(stage_v.at[pl.ds(0, 1280)], acc.at[pl.ds(s * 1280, 1280)])
    plsc.subcore_barrier()

    @pl.when(c == 0)
    def _():
        for half in range(2):
            base = (half * NS + s) * E_CH
            @pl.loop(0, E_CH // B_E)
            def _(bj):
                pltpu.sync_copy(head2.at[pl.ds(base + bj * B_E, B_E)], idx_v)
                @pl.loop(0, B_E)
                def _(j):
                    pltpu.sync_copy(ones_v, acc.at[idx_v.at[j]], add=True)
    @pl.when(c == 1)
    def _():
        for half in range(2):
            base = (half * NS + s) * I_CH
            @pl.loop(0, I_CH // B_E)
            def _(bj):
                pltpu.sync_copy(cols2.at[pl.ds(base + bj * B_E, B_E)], idx_v)
                @pl.loop(0, B_E)
                def _(j):
                    pltpu.sync_copy(ones_v, acc.at[idx_v.at[j]], add=True)
            rem = I_CH % B_E
            rbase = base + (I_CH // B_E) * B_E
            pltpu.sync_copy(cols2.at[pl.ds(rbase, rem)], idx_v.at[pl.ds(0, rem)])
            @pl.loop(0, rem)
            def _(j):
                pltpu.sync_copy(ones_v, acc.at[idx_v.at[j]], add=True)
    plsc.subcore_barrier()

    @pl.when(c == 0)
    def _():
        pltpu.sync_copy(acc.at[pl.ds(s * 3200, 3200)], stage_v)
        pltpu.sync_copy(stage_v, cnt_e.at[pl.ds(s * 3200, 3200)])
    @pl.when(c == 1)
    def _():
        pltpu.sync_copy(acc.at[pl.ds(s * 1280, 1280)], stage_v.at[pl.ds(0, 1280)])
        pltpu.sync_copy(stage_v.at[pl.ds(0, 1280)], cnt_i.at[pl.ds(s * 1280, 1280)])


@functools.partial(
    pl.kernel,
    out_type=(jax.ShapeDtypeStruct((ENT_ACC, HALF), jnp.float32),
              jax.ShapeDtypeStruct((ENT_ACC, HALF), jnp.float32)),
    mesh=_mesh,
    compiler_params=_sc_params,
    scratch_types=(
        pltpu.VMEM_SHARED((ENT_ACC, HALF), jnp.float32),
        pltpu.VMEM((B_E, CHUNK), jnp.int32),
        pltpu.VMEM((B_E, CHUNK), jnp.int32),
        pltpu.VMEM((B_E, CHUNK), jnp.int32),
        pltpu.VMEM((CHUNK, HALF), jnp.float32),
        pltpu.VMEM((CHUNK, HALF), jnp.float32),
        pltpu.VMEM((CHUNK, HALF), jnp.float32),
        pltpu.SemaphoreType.DMA,
        pltpu.SemaphoreType.DMA,
    ),
)
def _sc_kg_agg(e_lo, e_hi, w_lo, w_hi, tail2, head2, type2,
               out_lo, out_hi,
               acc, tail_v, head_v, type_v, gbuf, wbuf, fbuf, sem0, sem1):
    c = lax.axis_index("c")
    s = lax.axis_index("s")
    _zero_rows(fbuf, CHUNK)
    rbase = s * (ENT_ACC // NS)
    @pl.loop(0, ENT_ACC // NS // CHUNK)
    def _(k):
        pltpu.sync_copy(fbuf, acc.at[pl.ds(rbase + k * CHUNK, CHUNK)])
    plsc.subcore_barrier()

    for core in range(NC):
        tab = (e_lo, e_hi)[core]
        wt = (w_lo, w_hi)[core]
        @pl.when(c == core)
        def _():
          for half in range(2):
            slab = (half * NS + s) * E_CH
            @pl.loop(0, E_CH // B_E)
            def _(bj):
                base = slab + bj * B_E
                pltpu.sync_copy(tail2.at[pl.ds(base, B_E)], tail_v)
                pltpu.sync_copy(head2.at[pl.ds(base, B_E)], head_v)
                pltpu.sync_copy(type2.at[pl.ds(base, B_E)], type_v)
                @pl.loop(0, B_E)
                def _(j):
                    cp1 = pltpu.async_copy(tab.at[tail_v.at[j]], gbuf, sem0)
                    cp2 = pltpu.async_copy(wt.at[type_v.at[j]], wbuf, sem1)
                    cp1.wait()
                    cp2.wait()
                    @pl.loop(0, CHUNK, unroll=8)
                    def _(r):
                        gbuf[r, pl.ds(0, L)] = gbuf[r, pl.ds(0, L)] * wbuf[r, pl.ds(0, L)]
                        gbuf[r, pl.ds(L, L)] = gbuf[r, pl.ds(L, L)] * wbuf[r, pl.ds(L, L)]
                    pltpu.sync_copy(gbuf, acc.at[head_v.at[j]], add=True)
    plsc.subcore_barrier()

    for core in range(NC):
        outp = (out_lo, out_hi)[core]
        @pl.when(c == core)
        def _():
            @pl.loop(0, ENT_ACC // NS // CHUNK)
            def _(k):
                pltpu.sync_copy(acc.at[pl.ds(rbase + k * CHUNK, CHUNK)], fbuf)
                pltpu.sync_copy(fbuf, outp.at[pl.ds(rbase + k * CHUNK, CHUNK)])


@functools.partial(
    pl.kernel,
    out_type=(jax.ShapeDtypeStruct((ITEM_ACC, HALF), jnp.float32),
              jax.ShapeDtypeStruct((ITEM_ACC, HALF), jnp.float32)),
    mesh=_mesh,
    compiler_params=_sc_params,
    scratch_types=(
        pltpu.VMEM_SHARED((ITEM_ACC, HALF), jnp.float32),
        pltpu.VMEM((B_I, CHUNK), jnp.int32),
        pltpu.VMEM((B_I, CHUNK), jnp.int32),
        pltpu.VMEM((CHUNK, HALF), jnp.float32),
        pltpu.VMEM((1, HALF), jnp.float32),
        pltpu.VMEM((CHUNK, HALF), jnp.float32),
        pltpu.SemaphoreType.DMA,
    ),
)
def _sc_iu_agg(u_lo, u_hi, w0_lo, w0_hi, rowg2, cols2,
               out_lo, out_hi,
               acc, row_v, col_v, gbuf, wrow, fbuf, sem0):
    c = lax.axis_index("c")
    s = lax.axis_index("s")
    _zero_rows(fbuf, CHUNK)
    rbase = s * (ITEM_ACC // NS)
    @pl.loop(0, ITEM_ACC // NS // CHUNK)
    def _(k):
        pltpu.sync_copy(fbuf, acc.at[pl.ds(rbase + k * CHUNK, CHUNK)])
    plsc.subcore_barrier()

    for core in range(NC):
        tab = (u_lo, u_hi)[core]
        w0 = (w0_lo, w0_hi)[core]
        @pl.when(c == core)
        def _():
          pltpu.sync_copy(w0, wrow)
          wa = wrow[0, pl.ds(0, L)]
          wb = wrow[0, pl.ds(L, L)]
          for half in range(2):
            slab = (half * NS + s) * I_CH
            @pl.loop(0, I_CH // B_I)
            def _(bj):
                base = slab + bj * B_I
                pltpu.sync_copy(rowg2.at[pl.ds(base, B_I)], row_v)
                pltpu.sync_copy(cols2.at[pl.ds(base, B_I)], col_v)
                @pl.loop(0, B_I)
                def _(j):
                    pltpu.async_copy(tab.at[row_v.at[j]], gbuf, sem0).wait()
                    @pl.loop(0, CHUNK, unroll=8)
                    def _(r):
                        gbuf[r, pl.ds(0, L)] = gbuf[r, pl.ds(0, L)] * wa
                        gbuf[r, pl.ds(L, L)] = gbuf[r, pl.ds(L, L)] * wb
                    pltpu.sync_copy(gbuf, acc.at[col_v.at[j]], add=True)
    plsc.subcore_barrier()

    for core in range(NC):
        outp = (out_lo, out_hi)[core]
        @pl.when(c == core)
        def _():
            @pl.loop(0, ITEM_ACC // NS // CHUNK)
            def _(k):
                pltpu.sync_copy(acc.at[pl.ds(rbase + k * CHUNK, CHUNK)], fbuf)
                pltpu.sync_copy(fbuf, outp.at[pl.ds(rbase + k * CHUNK, CHUNK)])


@functools.partial(
    pl.kernel,
    out_type=(jax.ShapeDtypeStruct((USER_ACC, HALF), jnp.float32),
              jax.ShapeDtypeStruct((USER_ACC, HALF), jnp.float32)),
    mesh=_mesh,
    compiler_params=_sc_params,
    scratch_types=(
        pltpu.VMEM_SHARED((USER_ACC, HALF), jnp.float32),
        pltpu.VMEM((B_I, CHUNK), jnp.int32),
        pltpu.VMEM((B_I, CHUNK), jnp.int32),
        pltpu.VMEM((CHUNK, HALF), jnp.float32),
        pltpu.VMEM((CHUNK, HALF), jnp.float32),
        pltpu.SemaphoreType.DMA,
    ),
)
def _sc_user_agg(f_lo, f_hi, colg2, rows2,
                 out_lo, out_hi,
                 acc, col_v, row_v, gbuf, fbuf, sem0):
    c = lax.axis_index("c")
    s = lax.axis_index("s")
    _zero_rows(fbuf, CHUNK)
    rbase = s * (USER_ACC // NS)
    @pl.loop(0, USER_ACC // NS // CHUNK)
    def _(k):
        pltpu.sync_copy(fbuf, acc.at[pl.ds(rbase + k * CHUNK, CHUNK)])
    plsc.subcore_barrier()

    for core in range(NC):
        tab = (f_lo, f_hi)[core]
        @pl.when(c == core)
        def _():
          for half in range(2):
            slab = (half * NS + s) * I_CH
            @pl.loop(0, I_CH // B_I)
            def _(bj):
                base = slab + bj * B_I
                pltpu.sync_copy(colg2.at[pl.ds(base, B_I)], col_v)
                pltpu.sync_copy(rows2.at[pl.ds(base, B_I)], row_v)
                @pl.loop(0, B_I)
                def _(j):
                    pltpu.async_copy(tab.at[col_v.at[j]], gbuf, sem0).wait()
                    pltpu.sync_copy(gbuf, acc.at[row_v.at[j]], add=True)
    plsc.subcore_barrier()

    for core in range(NC):
        outp = (out_lo, out_hi)[core]
        @pl.when(c == core)
        def _():
            @pl.loop(0, USER_ACC // NS // CHUNK)
            def _(k):
                pltpu.sync_copy(acc.at[pl.ds(rbase + k * CHUNK, CHUNK)], fbuf)
                pltpu.sync_copy(fbuf, outp.at[pl.ds(rbase + k * CHUNK, CHUNK)])


def _tc_gate(agg_lo, agg_hi, cnt_e, iu_lo, iu_hi, cnt_i, g1t, g2t, res_prev):
    blk = 1000

    def body(alo, ahi, ce, ilo, ihi, ci, g1, g2, rp,
             flo, fhi, elo, ehi, rout):
        ikg = jnp.concatenate([alo[...], ahi[...]], axis=1) / jnp.maximum(ce[...], 1.0)
        iu = jnp.concatenate([ilo[...], ihi[...]], axis=1) / jnp.maximum(ci[...], 1.0)
        z = (jnp.dot(ikg, g1[...], preferred_element_type=jnp.float32)
             + jnp.dot(iu, g2[...], preferred_element_type=jnp.float32))
        gi = jax.nn.sigmoid(z)
        f = gi * ikg + (1.0 - gi) * iu
        flo[...] = f[:, :HALF]
        fhi[...] = f[:, HALF:]
        n = jnp.sqrt(jnp.sum(f * f, axis=1, keepdims=True))
        fn = f / jnp.maximum(n, 1e-12)
        elo[...] = fn[:, :HALF]
        ehi[...] = fn[:, HALF:]
        rout[...] = rp[...] + fn

    half_spec = pl.BlockSpec((blk, HALF), lambda i: (i, 0))
    cnt_spec = pl.BlockSpec((blk, 1), lambda i: (i, 0))
    mat_spec = pl.BlockSpec((DIM, DIM), lambda i: (0, 0))
    full_spec = pl.BlockSpec((blk, DIM), lambda i: (i, 0))
    return pl.pallas_call(
        body,
        grid=(N_ITEMS // blk,),
        in_specs=[half_spec, half_spec, cnt_spec, half_spec, half_spec,
                  cnt_spec, mat_spec, mat_spec, full_spec],
        out_specs=[half_spec, half_spec, half_spec, half_spec, full_spec],
        out_shape=[
            jax.ShapeDtypeStruct((N_ITEMS, HALF), jnp.float32),
            jax.ShapeDtypeStruct((N_ITEMS, HALF), jnp.float32),
            jax.ShapeDtypeStruct((N_ITEMS, HALF), jnp.float32),
            jax.ShapeDtypeStruct((N_ITEMS, HALF), jnp.float32),
            jax.ShapeDtypeStruct((N_ITEMS, DIM), jnp.float32),
        ],
    )(agg_lo, agg_hi, cnt_e, iu_lo, iu_hi, cnt_i, g1t, g2t, res_prev)


def _tc_normres(x_lo, x_hi, res_prev):
    n_rows = x_lo.shape[0]
    blk = 1000

    def body(xlo, xhi, rp, nlo, nhi, rout):
        x = jnp.concatenate([xlo[...], xhi[...]], axis=1)
        n = jnp.sqrt(jnp.sum(x * x, axis=1, keepdims=True))
        xn = x / jnp.maximum(n, 1e-12)
        nlo[...] = xn[:, :HALF]
        nhi[...] = xn[:, HALF:]
        rout[...] = rp[...] + xn

    half_spec = pl.BlockSpec((blk, HALF), lambda i: (i, 0))
    full_spec = pl.BlockSpec((blk, DIM), lambda i: (i, 0))
    return pl.pallas_call(
        body,
        grid=(n_rows // blk,),
        in_specs=[half_spec, half_spec, full_spec],
        out_specs=[half_spec, half_spec, full_spec],
        out_shape=[
            jax.ShapeDtypeStruct((n_rows, HALF), jnp.float32),
            jax.ShapeDtypeStruct((n_rows, HALF), jnp.float32),
            jax.ShapeDtypeStruct((n_rows, DIM), jnp.float32),
        ],
    )(x_lo, x_hi, res_prev)


def _pack(x, nch, padval):
    tot = NW * nch * CHUNK
    return jnp.pad(x.astype(jnp.int32), (0, tot - x.shape[0]),
                   constant_values=padval).reshape(NW * nch, CHUNK)


def kernel(user_emb, entity_emb, edge_index, edge_type, mat_row, mat_col, mat_val,
           weight, gate1_w0, gate2_w0, gate1_w1, gate2_w1):
    head = edge_index[0]
    tail = edge_index[1]
    tail2 = _pack(tail, E_CH, 0)
    head2 = _pack(head, E_CH, N_ENTITIES)
    type2 = _pack(edge_type, E_CH, 0)
    rowg2 = _pack(mat_row, I_CH, 0)
    rows2 = _pack(mat_row, I_CH, N_USERS)
    colg2 = _pack(mat_col, I_CH, 0)
    cols2 = _pack(mat_col, I_CH, N_ITEMS)

    cnt_e_raw, cnt_i_raw = _sc_counts(head2, cols2)
    cnt_e = cnt_e_raw[:N_ITEMS].reshape(N_ITEMS, 1)
    cnt_i = cnt_i_raw[:N_ITEMS].reshape(N_ITEMS, 1)

    e_lo, e_hi = entity_emb[:, :HALF], entity_emb[:, HALF:]
    u_lo, u_hi = user_emb[:, :HALF], user_emb[:, HALF:]
    w_lo, w_hi = weight[:, :HALF], weight[:, HALF:]
    w0_lo, w0_hi = weight[0:1, :HALF], weight[0:1, HALF:]
    g1t = (gate1_w0.T, gate1_w1.T)
    g2t = (gate2_w0.T, gate2_w1.T)

    res_i = entity_emb[:N_ITEMS]
    res_a = entity_emb[N_ITEMS:]
    res_u = user_emb

    for i in range(N_HOPS):
        agg_lo, agg_hi = _sc_kg_agg(e_lo, e_hi, w_lo, w_hi, tail2, head2, type2)
        iu_lo, iu_hi = _sc_iu_agg(u_lo, u_hi, w0_lo, w0_hi, rowg2, cols2)
        f_lo, f_hi, en_lo, en_hi, res_i = _tc_gate(
            agg_lo[:N_ITEMS], agg_hi[:N_ITEMS], cnt_e,
            iu_lo[:N_ITEMS], iu_hi[:N_ITEMS], cnt_i, g1t[i], g2t[i], res_i)
        us_lo, us_hi = _sc_user_agg(f_lo, f_hi, colg2, rows2)
        an_lo, an_hi, res_a = _tc_normres(
            agg_lo[N_ITEMS:N_ENTITIES], agg_hi[N_ITEMS:N_ENTITIES], res_a)
        un_lo, un_hi, res_u = _tc_normres(us_lo[:N_USERS], us_hi[:N_USERS], res_u)
        if i + 1 < N_HOPS:
            e_lo = jnp.concatenate([en_lo, an_lo], axis=0)
            e_hi = jnp.concatenate([en_hi, an_hi], axis=0)
            u_lo, u_hi = un_lo, un_hi

    entity_res = jnp.concatenate([res_i, res_a], axis=0)
    return (entity_res, res_u)


# speedup vs baseline: 3.9288x; 1.4707x over previous
import functools

import jax
import jax.numpy as jnp
from jax import lax
from jax.experimental import pallas as pl
from jax.experimental.pallas import tpu as pltpu
from jax.experimental.pallas import tpu_sc as plsc

N_USERS = 30000
N_ITEMS = 20000
N_ENTITIES = 50000
N_RELATIONS = 16
DIM = 64
HALF = 32
N_HOPS = 2
N_EDGES = 800000
N_INTER = 500000

NC = 2
NS = 16
NW = NC * NS
L = 16
CHUNK = 128

E_CH = 196
I_CH = 128
B_E = 7
B_I = 8

ENT_ACC = 50176
ITEM_ACC = 20480
USER_ACC = 30720

_mesh = plsc.VectorSubcoreMesh(
    core_axis_name="c", subcore_axis_name="s", num_cores=NC, num_subcores=NS)
_sc_params = pltpu.CompilerParams(use_tc_tiling_on_sc=False)


def _zero_rows(buf, nrows):
    @pl.loop(0, nrows, unroll=8)
    def _(r):
        z = jnp.zeros((L,), jnp.float32)
        buf[r, pl.ds(0, L)] = z
        buf[r, pl.ds(L, L)] = z


def _zero_flat(buf, n):
    @pl.loop(0, n // L, unroll=8)
    def _(k):
        buf[pl.ds(k * L, L)] = jnp.zeros((L,), jnp.float32)


@functools.partial(
    pl.kernel,
    out_type=(jax.ShapeDtypeStruct((ENT_ACC,), jnp.float32),
              jax.ShapeDtypeStruct((ITEM_ACC,), jnp.float32)),
    mesh=_mesh,
    compiler_params=_sc_params,
    scratch_types=(
        pltpu.VMEM_SHARED((ENT_ACC,), jnp.float32),
        pltpu.VMEM((B_E, CHUNK), jnp.int32),
        pltpu.VMEM((CHUNK,), jnp.float32),
        pltpu.VMEM((3136,), jnp.float32),
    ),
)
def _sc_counts(head2, cols2, cnt_e, cnt_i, acc, idx_v, ones_v, stage_v):
    c = lax.axis_index("c")
    s = lax.axis_index("s")
    @pl.loop(0, CHUNK // L, unroll=8)
    def _(k):
        ones_v[pl.ds(k * L, L)] = jnp.ones((L,), jnp.float32)
    _zero_flat(stage_v, 3136)

    @pl.when(c == 0)
    def _():
        pltpu.sync_copy(stage_v, acc.at[pl.ds(s * 3136, 3136)])
    @pl.when(c == 1)
    def _():
        pltpu.sync_copy(stage_v.at[pl.ds(0, 1280)], acc.at[pl.ds(s * 1280, 1280)])
    plsc.subcore_barrier()

    @pl.when(c == 0)
    def _():
        for half in range(2):
            base = (half * NS + s) * E_CH
            @pl.loop(0, E_CH // B_E)
            def _(bj):
                pltpu.sync_copy(head2.at[pl.ds(base + bj * B_E, B_E)], idx_v)
                @pl.loop(0, B_E)
                def _(j):
                    pltpu.sync_copy(ones_v, acc.at[idx_v.at[j]], add=True)
    @pl.when(c == 1)
    def _():
        for half in range(2):
            base = (half * NS + s) * I_CH
            @pl.loop(0, I_CH // B_E)
            def _(bj):
                pltpu.sync_copy(cols2.at[pl.ds(base + bj * B_E, B_E)], idx_v)
                @pl.loop(0, B_E)
                def _(j):
                    pltpu.sync_copy(ones_v, acc.at[idx_v.at[j]], add=True)
            rem = I_CH % B_E
            rbase = base + (I_CH // B_E) * B_E
            pltpu.sync_copy(cols2.at[pl.ds(rbase, rem)], idx_v.at[pl.ds(0, rem)])
            @pl.loop(0, rem)
            def _(j):
                pltpu.sync_copy(ones_v, acc.at[idx_v.at[j]], add=True)
    plsc.subcore_barrier()

    @pl.when(c == 0)
    def _():
        pltpu.sync_copy(acc.at[pl.ds(s * 3136, 3136)], stage_v)
        pltpu.sync_copy(stage_v, cnt_e.at[pl.ds(s * 3136, 3136)])
    @pl.when(c == 1)
    def _():
        pltpu.sync_copy(acc.at[pl.ds(s * 1280, 1280)], stage_v.at[pl.ds(0, 1280)])
        pltpu.sync_copy(stage_v.at[pl.ds(0, 1280)], cnt_i.at[pl.ds(s * 1280, 1280)])


@functools.partial(
    pl.kernel,
    out_type=(jax.ShapeDtypeStruct((ENT_ACC, HALF), jnp.float32),
              jax.ShapeDtypeStruct((ENT_ACC, HALF), jnp.float32)),
    mesh=_mesh,
    compiler_params=_sc_params,
    scratch_types=(
        pltpu.VMEM_SHARED((ENT_ACC, HALF), jnp.float32),
        pltpu.VMEM_SHARED((N_RELATIONS, HALF), jnp.float32),
        pltpu.VMEM((B_E, CHUNK), jnp.int32),
        pltpu.VMEM((B_E, CHUNK), jnp.int32),
        pltpu.VMEM((B_E, CHUNK), jnp.int32),
        pltpu.VMEM((CHUNK, HALF), jnp.float32),
        pltpu.VMEM((CHUNK, HALF), jnp.float32),
        pltpu.VMEM((CHUNK, HALF), jnp.float32),
        pltpu.VMEM((CHUNK, HALF), jnp.float32),
        pltpu.VMEM((CHUNK, HALF), jnp.float32),
        pltpu.VMEM((CHUNK, HALF), jnp.float32),
        pltpu.SemaphoreType.DMA, pltpu.SemaphoreType.DMA,
        pltpu.SemaphoreType.DMA, pltpu.SemaphoreType.DMA,
        pltpu.SemaphoreType.DMA, pltpu.SemaphoreType.DMA,
    ),
)
def _sc_kg_agg(e_lo, e_hi, w_lo, w_hi, tail2, head2, type2,
               out_lo, out_hi,
               acc, w_sp, tail_v, head_v, type_v,
               g0, g1, w0, w1, sb0, sb1,
               sg0, sg1, sw0, sw1, ss0, ss1):
    c = lax.axis_index("c")
    s = lax.axis_index("s")
    gb = (g0, g1)
    wb = (w0, w1)
    sb = (sb0, sb1)
    gsem = (sg0, sg1)
    wsem = (sw0, sw1)
    ssem = (ss0, ss1)

    _zero_rows(sb0, CHUNK)
    rbase = s * (ENT_ACC // NS)
    @pl.loop(0, 28)
    def _(k):
        pltpu.sync_copy(sb0.at[pl.ds(0, 112)], acc.at[pl.ds(rbase + k * 112, 112)])
    for core in range(NC):
        @pl.when((c == core) & (s == 0))
        def _():
            pltpu.sync_copy((w_lo, w_hi)[core], sb1.at[pl.ds(0, N_RELATIONS)])
            pltpu.sync_copy(sb1.at[pl.ds(0, N_RELATIONS)], w_sp)
    plsc.subcore_barrier()

    for core in range(NC):
        tab = (e_lo, e_hi)[core]
        @pl.when(c == core)
        def _():
          for half in range(2):
            slab = (half * NS + s) * E_CH
            @pl.loop(0, E_CH // B_E)
            def _(bj):
                base = slab + bj * B_E
                pltpu.sync_copy(tail2.at[pl.ds(base, B_E)], tail_v)
                pltpu.sync_copy(head2.at[pl.ds(base, B_E)], head_v)
                pltpu.sync_copy(type2.at[pl.ds(base, B_E)], type_v)
                dg = {}
                dw = {}
                dsc = {}
                for j in range(2):
                    dg[j] = pltpu.async_copy(tab.at[tail_v.at[j]], gb[j], gsem[j])
                    dw[j] = pltpu.async_copy(w_sp.at[type_v.at[j]], wb[j], wsem[j])
                for j in range(B_E):
                    sl = j % 2
                    dg[sl].wait()
                    dw[sl].wait()
                    if j >= 2:
                        dsc[sl].wait()
                    @pl.loop(0, CHUNK, unroll=8)
                    def _(r):
                        sb[sl][r, pl.ds(0, L)] = gb[sl][r, pl.ds(0, L)] * wb[sl][r, pl.ds(0, L)]
                        sb[sl][r, pl.ds(L, L)] = gb[sl][r, pl.ds(L, L)] * wb[sl][r, pl.ds(L, L)]
                    dsc[sl] = pltpu.async_copy(sb[sl], acc.at[head_v.at[j]], ssem[sl], add=True)
                    if j + 2 < B_E:
                        dg[sl] = pltpu.async_copy(tab.at[tail_v.at[j + 2]], gb[sl], gsem[sl])
                        dw[sl] = pltpu.async_copy(w_sp.at[type_v.at[j + 2]], wb[sl], wsem[sl])
                dsc[0].wait()
                dsc[1].wait()
    plsc.subcore_barrier()

    for core in range(NC):
        outp = (out_lo, out_hi)[core]
        @pl.when(c == core)
        def _():
            @pl.loop(0, 28)
            def _(k):
                pltpu.sync_copy(acc.at[pl.ds(rbase + k * 112, 112)], sb0.at[pl.ds(0, 112)])
                pltpu.sync_copy(sb0.at[pl.ds(0, 112)], outp.at[pl.ds(rbase + k * 112, 112)])


@functools.partial(
    pl.kernel,
    out_type=(jax.ShapeDtypeStruct((ITEM_ACC, HALF), jnp.float32),
              jax.ShapeDtypeStruct((ITEM_ACC, HALF), jnp.float32)),
    mesh=_mesh,
    compiler_params=_sc_params,
    scratch_types=(
        pltpu.VMEM_SHARED((ITEM_ACC, HALF), jnp.float32),
        pltpu.VMEM((B_I, CHUNK), jnp.int32),
        pltpu.VMEM((B_I, CHUNK), jnp.int32),
        pltpu.VMEM((CHUNK, HALF), jnp.float32),
        pltpu.VMEM((CHUNK, HALF), jnp.float32),
        pltpu.VMEM((CHUNK, HALF), jnp.float32),
        pltpu.VMEM((CHUNK, HALF), jnp.float32),
        pltpu.VMEM((1, HALF), jnp.float32),
        pltpu.SemaphoreType.DMA, pltpu.SemaphoreType.DMA,
        pltpu.SemaphoreType.DMA, pltpu.SemaphoreType.DMA,
    ),
)
def _sc_iu_agg(u_lo, u_hi, w0_lo, w0_hi, rowg2, cols2,
               out_lo, out_hi,
               acc, row_v, col_v, g0, g1, sb0, sb1, wrow,
               sg0, sg1, ss0, ss1):
    c = lax.axis_index("c")
    s = lax.axis_index("s")
    gb = (g0, g1)
    sb = (sb0, sb1)
    gsem = (sg0, sg1)
    ssem = (ss0, ss1)

    _zero_rows(sb0, CHUNK)
    rbase = s * (ITEM_ACC // NS)
    @pl.loop(0, ITEM_ACC // NS // CHUNK)
    def _(k):
        pltpu.sync_copy(sb0.at[pl.ds(0, CHUNK)], acc.at[pl.ds(rbase + k * CHUNK, CHUNK)])
    plsc.subcore_barrier()

    for core in range(NC):
        tab = (u_lo, u_hi)[core]
        w0t = (w0_lo, w0_hi)[core]
        @pl.when(c == core)
        def _():
          pltpu.sync_copy(w0t, wrow)
          wa = wrow[0, pl.ds(0, L)]
          wvb = wrow[0, pl.ds(L, L)]
          for half in range(2):
            slab = (half * NS + s) * I_CH
            @pl.loop(0, I_CH // B_I)
            def _(bj):
                base = slab + bj * B_I
                pltpu.sync_copy(rowg2.at[pl.ds(base, B_I)], row_v)
                pltpu.sync_copy(cols2.at[pl.ds(base, B_I)], col_v)
                dg = {}
                dsc = {}
                for j in range(2):
                    dg[j] = pltpu.async_copy(tab.at[row_v.at[j]], gb[j], gsem[j])
                for j in range(B_I):
                    sl = j % 2
                    dg[sl].wait()
                    if j >= 2:
                        dsc[sl].wait()
                    @pl.loop(0, CHUNK, unroll=8)
                    def _(r):
                        sb[sl][r, pl.ds(0, L)] = gb[sl][r, pl.ds(0, L)] * wa
                        sb[sl][r, pl.ds(L, L)] = gb[sl][r, pl.ds(L, L)] * wvb
                    dsc[sl] = pltpu.async_copy(sb[sl], acc.at[col_v.at[j]], ssem[sl], add=True)
                    if j + 2 < B_I:
                        dg[sl] = pltpu.async_copy(tab.at[row_v.at[j + 2]], gb[sl], gsem[sl])
                dsc[0].wait()
                dsc[1].wait()
    plsc.subcore_barrier()

    for core in range(NC):
        outp = (out_lo, out_hi)[core]
        @pl.when(c == core)
        def _():
            @pl.loop(0, ITEM_ACC // NS // CHUNK)
            def _(k):
                pltpu.sync_copy(acc.at[pl.ds(rbase + k * CHUNK, CHUNK)], sb0.at[pl.ds(0, CHUNK)])
                pltpu.sync_copy(sb0.at[pl.ds(0, CHUNK)], outp.at[pl.ds(rbase + k * CHUNK, CHUNK)])


@functools.partial(
    pl.kernel,
    out_type=(jax.ShapeDtypeStruct((USER_ACC, HALF), jnp.float32),
              jax.ShapeDtypeStruct((USER_ACC, HALF), jnp.float32)),
    mesh=_mesh,
    compiler_params=_sc_params,
    scratch_types=(
        pltpu.VMEM_SHARED((USER_ACC, HALF), jnp.float32),
        pltpu.VMEM((B_I, CHUNK), jnp.int32),
        pltpu.VMEM((B_I, CHUNK), jnp.int32),
        pltpu.VMEM((CHUNK, HALF), jnp.float32),
        pltpu.VMEM((CHUNK, HALF), jnp.float32),
        pltpu.VMEM((CHUNK, HALF), jnp.float32),
        pltpu.VMEM((CHUNK, HALF), jnp.float32),
        pltpu.SemaphoreType.DMA, pltpu.SemaphoreType.DMA,
        pltpu.SemaphoreType.DMA, pltpu.SemaphoreType.DMA,
        pltpu.SemaphoreType.DMA, pltpu.SemaphoreType.DMA,
        pltpu.SemaphoreType.DMA, pltpu.SemaphoreType.DMA,
    ),
)
def _sc_user_agg(f_lo, f_hi, colg2, rows2,
                 out_lo, out_hi,
                 acc, col_v, row_v, g0, g1, g2, g3,
                 sg0, sg1, sg2, sg3, ss0, ss1, ss2, ss3):
    c = lax.axis_index("c")
    s = lax.axis_index("s")
    gb = (g0, g1, g2, g3)
    gsem = (sg0, sg1, sg2, sg3)
    ssem = (ss0, ss1, ss2, ss3)

    _zero_rows(g0, CHUNK)
    rbase = s * (USER_ACC // NS)
    @pl.loop(0, USER_ACC // NS // CHUNK)
    def _(k):
        pltpu.sync_copy(g0.at[pl.ds(0, CHUNK)], acc.at[pl.ds(rbase + k * CHUNK, CHUNK)])
    plsc.subcore_barrier()

    for core in range(NC):
        tab = (f_lo, f_hi)[core]
        @pl.when(c == core)
        def _():
          for half in range(2):
            slab = (half * NS + s) * I_CH
            @pl.loop(0, I_CH // B_I)
            def _(bj):
                base = slab + bj * B_I
                pltpu.sync_copy(colg2.at[pl.ds(base, B_I)], col_v)
                pltpu.sync_copy(rows2.at[pl.ds(base, B_I)], row_v)
                dg = {}
                dsc = {}
                for j in range(2):
                    dg[j] = pltpu.async_copy(tab.at[col_v.at[j]], gb[j], gsem[j])
                for j in range(B_I):
                    sl = j % 4
                    dg[sl].wait()
                    dsc[sl] = pltpu.async_copy(gb[sl], acc.at[row_v.at[j]], ssem[sl], add=True)
                    if j + 2 < B_I:
                        tsl = (j + 2) % 4
                        if j >= 2:
                            dsc[tsl].wait()
                        dg[tsl] = pltpu.async_copy(tab.at[col_v.at[j + 2]], gb[tsl], gsem[tsl])
                for j in range(B_I - 4, B_I):
                    dsc[j % 4].wait()
    plsc.subcore_barrier()

    for core in range(NC):
        outp = (out_lo, out_hi)[core]
        @pl.when(c == core)
        def _():
            @pl.loop(0, USER_ACC // NS // CHUNK)
            def _(k):
                pltpu.sync_copy(acc.at[pl.ds(rbase + k * CHUNK, CHUNK)], g0.at[pl.ds(0, CHUNK)])
                pltpu.sync_copy(g0.at[pl.ds(0, CHUNK)], outp.at[pl.ds(rbase + k * CHUNK, CHUNK)])


def _tc_gate(agg_lo, agg_hi, cnt_e, iu_lo, iu_hi, cnt_i, g1t, g2t, res_prev):
    blk = 1000

    def body(alo, ahi, ce, ilo, ihi, ci, g1, g2, rp,
             flo, fhi, elo, ehi, rout):
        ikg = jnp.concatenate([alo[...], ahi[...]], axis=1) / jnp.maximum(ce[...], 1.0)
        iu = jnp.concatenate([ilo[...], ihi[...]], axis=1) / jnp.maximum(ci[...], 1.0)
        z = (jnp.dot(ikg, g1[...], preferred_element_type=jnp.float32)
             + jnp.dot(iu, g2[...], preferred_element_type=jnp.float32))
        gi = jax.nn.sigmoid(z)
        f = gi * ikg + (1.0 - gi) * iu
        flo[...] = f[:, :HALF]
        fhi[...] = f[:, HALF:]
        n = jnp.sqrt(jnp.sum(f * f, axis=1, keepdims=True))
        fn = f / jnp.maximum(n, 1e-12)
        elo[...] = fn[:, :HALF]
        ehi[...] = fn[:, HALF:]
        rout[...] = rp[...] + fn

    half_spec = pl.BlockSpec((blk, HALF), lambda i: (i, 0))
    cnt_spec = pl.BlockSpec((blk, 1), lambda i: (i, 0))
    mat_spec = pl.BlockSpec((DIM, DIM), lambda i: (0, 0))
    full_spec = pl.BlockSpec((blk, DIM), lambda i: (i, 0))
    return pl.pallas_call(
        body,
        grid=(N_ITEMS // blk,),
        in_specs=[half_spec, half_spec, cnt_spec, half_spec, half_spec,
                  cnt_spec, mat_spec, mat_spec, full_spec],
        out_specs=[half_spec, half_spec, half_spec, half_spec, full_spec],
        out_shape=[
            jax.ShapeDtypeStruct((N_ITEMS, HALF), jnp.float32),
            jax.ShapeDtypeStruct((N_ITEMS, HALF), jnp.float32),
            jax.ShapeDtypeStruct((N_ITEMS, HALF), jnp.float32),
            jax.ShapeDtypeStruct((N_ITEMS, HALF), jnp.float32),
            jax.ShapeDtypeStruct((N_ITEMS, DIM), jnp.float32),
        ],
    )(agg_lo, agg_hi, cnt_e, iu_lo, iu_hi, cnt_i, g1t, g2t, res_prev)


def _tc_normres(x_lo, x_hi, res_prev):
    n_rows = x_lo.shape[0]
    blk = 1000

    def body(xlo, xhi, rp, nlo, nhi, rout):
        x = jnp.concatenate([xlo[...], xhi[...]], axis=1)
        n = jnp.sqrt(jnp.sum(x * x, axis=1, keepdims=True))
        xn = x / jnp.maximum(n, 1e-12)
        nlo[...] = xn[:, :HALF]
        nhi[...] = xn[:, HALF:]
        rout[...] = rp[...] + xn

    half_spec = pl.BlockSpec((blk, HALF), lambda i: (i, 0))
    full_spec = pl.BlockSpec((blk, DIM), lambda i: (i, 0))
    return pl.pallas_call(
        body,
        grid=(n_rows // blk,),
        in_specs=[half_spec, half_spec, full_spec],
        out_specs=[half_spec, half_spec, full_spec],
        out_shape=[
            jax.ShapeDtypeStruct((n_rows, HALF), jnp.float32),
            jax.ShapeDtypeStruct((n_rows, HALF), jnp.float32),
            jax.ShapeDtypeStruct((n_rows, DIM), jnp.float32),
        ],
    )(x_lo, x_hi, res_prev)


def _pack(x, nch, padval):
    tot = NW * nch * CHUNK
    return jnp.pad(x.astype(jnp.int32), (0, tot - x.shape[0]),
                   constant_values=padval).reshape(NW * nch, CHUNK)


def kernel(user_emb, entity_emb, edge_index, edge_type, mat_row, mat_col, mat_val,
           weight, gate1_w0, gate2_w0, gate1_w1, gate2_w1):
    head = edge_index[0]
    tail = edge_index[1]
    tail2 = _pack(tail, E_CH, 0)
    head2 = _pack(head, E_CH, N_ENTITIES)
    type2 = _pack(edge_type, E_CH, 0)
    rowg2 = _pack(mat_row, I_CH, 0)
    rows2 = _pack(mat_row, I_CH, N_USERS)
    colg2 = _pack(mat_col, I_CH, 0)
    cols2 = _pack(mat_col, I_CH, N_ITEMS)

    cnt_e_raw, cnt_i_raw = _sc_counts(head2, cols2)
    cnt_e = cnt_e_raw[:N_ITEMS].reshape(N_ITEMS, 1)
    cnt_i = cnt_i_raw[:N_ITEMS].reshape(N_ITEMS, 1)

    e_lo, e_hi = entity_emb[:, :HALF], entity_emb[:, HALF:]
    u_lo, u_hi = user_emb[:, :HALF], user_emb[:, HALF:]
    w_lo, w_hi = weight[:, :HALF], weight[:, HALF:]
    w0_lo, w0_hi = weight[0:1, :HALF], weight[0:1, HALF:]
    g1t = (gate1_w0.T, gate1_w1.T)
    g2t = (gate2_w0.T, gate2_w1.T)

    res_i = entity_emb[:N_ITEMS]
    res_a = entity_emb[N_ITEMS:]
    res_u = user_emb

    for i in range(N_HOPS):
        agg_lo, agg_hi = _sc_kg_agg(e_lo, e_hi, w_lo, w_hi, tail2, head2, type2)
        iu_lo, iu_hi = _sc_iu_agg(u_lo, u_hi, w0_lo, w0_hi, rowg2, cols2)
        f_lo, f_hi, en_lo, en_hi, res_i = _tc_gate(
            agg_lo[:N_ITEMS], agg_hi[:N_ITEMS], cnt_e,
            iu_lo[:N_ITEMS], iu_hi[:N_ITEMS], cnt_i, g1t[i], g2t[i], res_i)
        us_lo, us_hi = _sc_user_agg(f_lo, f_hi, colg2, rows2)
        an_lo, an_hi, res_a = _tc_normres(
            agg_lo[N_ITEMS:N_ENTITIES], agg_hi[N_ITEMS:N_ENTITIES], res_a)
        un_lo, un_hi, res_u = _tc_normres(us_lo[:N_USERS], us_hi[:N_USERS], res_u)
        if i + 1 < N_HOPS:
            e_lo = jnp.concatenate([en_lo, an_lo], axis=0)
            e_hi = jnp.concatenate([en_hi, an_hi], axis=0)
            u_lo, u_hi = un_lo, un_hi

    entity_res = jnp.concatenate([res_i, res_a], axis=0)
    return (entity_res, res_u)


# speedup vs baseline: 4.4995x; 1.1453x over previous
import functools

import jax
import jax.numpy as jnp
from jax import lax
from jax.experimental import pallas as pl
from jax.experimental.pallas import tpu as pltpu
from jax.experimental.pallas import tpu_sc as plsc

N_USERS = 30000
N_ITEMS = 20000
N_ENTITIES = 50000
N_RELATIONS = 16
DIM = 64
HALF = 32
N_HOPS = 2
N_EDGES = 800000
N_INTER = 500000

NC = 2
NS = 16
NW = NC * NS
L = 16
CHUNK = 128

E_CH = 200
I_CH = 128
B_E = 8
B_I = 8

ENT_ACC = 50112
ENT_CNT = 50176
ITEM_ACC = 20480
USER_ACC = 30720

_mesh = plsc.VectorSubcoreMesh(
    core_axis_name="c", subcore_axis_name="s", num_cores=NC, num_subcores=NS)
_sc_params = pltpu.CompilerParams(use_tc_tiling_on_sc=False)


def _zero_rows(buf, nrows):
    @pl.loop(0, nrows, unroll=8)
    def _(r):
        z = jnp.zeros((L,), jnp.float32)
        buf[r, pl.ds(0, L)] = z
        buf[r, pl.ds(L, L)] = z


def _zero_flat(buf, n):
    @pl.loop(0, n // L, unroll=8)
    def _(k):
        buf[pl.ds(k * L, L)] = jnp.zeros((L,), jnp.float32)


@functools.partial(
    pl.kernel,
    out_type=(jax.ShapeDtypeStruct((ENT_CNT,), jnp.float32),
              jax.ShapeDtypeStruct((ITEM_ACC,), jnp.float32)),
    mesh=_mesh,
    compiler_params=_sc_params,
    scratch_types=(
        pltpu.VMEM_SHARED((ENT_CNT,), jnp.float32),
        pltpu.VMEM((B_E, CHUNK), jnp.int32),
        pltpu.VMEM((CHUNK,), jnp.float32),
        pltpu.VMEM((3136,), jnp.float32),
    ),
)
def _sc_counts(head2, cols2, cnt_e, cnt_i, acc, idx_v, ones_v, stage_v):
    c = lax.axis_index("c")
    s = lax.axis_index("s")
    @pl.loop(0, CHUNK // L, unroll=8)
    def _(k):
        ones_v[pl.ds(k * L, L)] = jnp.ones((L,), jnp.float32)
    _zero_flat(stage_v, 3136)

    @pl.when(c == 0)
    def _():
        pltpu.sync_copy(stage_v, acc.at[pl.ds(s * 3136, 3136)])
    @pl.when(c == 1)
    def _():
        pltpu.sync_copy(stage_v.at[pl.ds(0, 1280)], acc.at[pl.ds(s * 1280, 1280)])
    plsc.subcore_barrier()

    @pl.when(c == 0)
    def _():
        for half in range(2):
            base = (half * NS + s) * E_CH
            @pl.loop(0, E_CH // B_E)
            def _(bj):
                pltpu.sync_copy(head2.at[pl.ds(base + bj * B_E, B_E)], idx_v)
                @pl.loop(0, B_E)
                def _(j):
                    pltpu.sync_copy(ones_v, acc.at[idx_v.at[j]], add=True)
    @pl.when(c == 1)
    def _():
        for half in range(2):
            base = (half * NS + s) * I_CH
            @pl.loop(0, I_CH // B_E)
            def _(bj):
                pltpu.sync_copy(cols2.at[pl.ds(base + bj * B_E, B_E)], idx_v)
                @pl.loop(0, B_E)
                def _(j):
                    pltpu.sync_copy(ones_v, acc.at[idx_v.at[j]], add=True)
    plsc.subcore_barrier()

    @pl.when(c == 0)
    def _():
        pltpu.sync_copy(acc.at[pl.ds(s * 3136, 3136)], stage_v)
        pltpu.sync_copy(stage_v, cnt_e.at[pl.ds(s * 3136, 3136)])
    @pl.when(c == 1)
    def _():
        pltpu.sync_copy(acc.at[pl.ds(s * 1280, 1280)], stage_v.at[pl.ds(0, 1280)])
        pltpu.sync_copy(stage_v.at[pl.ds(0, 1280)], cnt_i.at[pl.ds(s * 1280, 1280)])


@functools.partial(
    pl.kernel,
    out_type=(jax.ShapeDtypeStruct((ENT_ACC, HALF), jnp.float32),
              jax.ShapeDtypeStruct((ENT_ACC, HALF), jnp.float32)),
    mesh=_mesh,
    compiler_params=_sc_params,
    scratch_types=(
        pltpu.VMEM_SHARED((ENT_ACC, HALF), jnp.float32),
        pltpu.VMEM_SHARED((N_RELATIONS, HALF), jnp.float32),
        pltpu.VMEM((B_E, CHUNK), jnp.int32),
        pltpu.VMEM((B_E, CHUNK), jnp.int32),
        pltpu.VMEM((B_E, CHUNK), jnp.int32),
        pltpu.VMEM((B_E, CHUNK), jnp.int32),
        pltpu.VMEM((B_E, CHUNK), jnp.int32),
        pltpu.VMEM((B_E, CHUNK), jnp.int32),
        pltpu.VMEM((CHUNK, HALF), jnp.float32),
        pltpu.VMEM((CHUNK, HALF), jnp.float32),
        pltpu.VMEM((CHUNK, HALF), jnp.float32),
        pltpu.VMEM((CHUNK, HALF), jnp.float32),
        pltpu.VMEM((CHUNK, HALF), jnp.float32),
        pltpu.VMEM((CHUNK, HALF), jnp.float32),
        pltpu.SemaphoreType.DMA, pltpu.SemaphoreType.DMA,
        pltpu.SemaphoreType.DMA, pltpu.SemaphoreType.DMA,
        pltpu.SemaphoreType.DMA, pltpu.SemaphoreType.DMA,
        pltpu.SemaphoreType.DMA,
    ),
)
def _sc_kg_agg(e_lo, e_hi, w_lo, w_hi, tail2, head2, type2,
               out_lo, out_hi,
               acc, w_sp, tail_v0, head_v0, type_v0, tail_v1, head_v1, type_v1,
               g0, g1, w0b, w1b, sb0, sb1,
               sg0, sg1, sw0, sw1, ss0, ss1, isem):
    c = lax.axis_index("c")
    s = lax.axis_index("s")
    gb = (g0, g1)
    wb = (w0b, w1b)
    sb = (sb0, sb1)
    gsem = (sg0, sg1)
    wsem = (sw0, sw1)
    ssem = (ss0, ss1)
    ivs = ((tail_v0, head_v0, type_v0), (tail_v1, head_v1, type_v1))
    NBLK = E_CH // B_E
    NBLK2 = 2 * NBLK

    _zero_rows(sb0, CHUNK)
    rbase = s * (ENT_ACC // NS)
    @pl.loop(0, 27)
    def _(k):
        pltpu.sync_copy(sb0.at[pl.ds(0, 116)], acc.at[pl.ds(rbase + k * 116, 116)])
    for core in range(NC):
        @pl.when((c == core) & (s == 0))
        def _():
            pltpu.sync_copy((w_lo, w_hi)[core], sb1.at[pl.ds(0, N_RELATIONS)])
            pltpu.sync_copy(sb1.at[pl.ds(0, N_RELATIONS)], w_sp)
    plsc.subcore_barrier()

    for core in range(NC):
        tab = (e_lo, e_hi)[core]
        @pl.when(c == core)
        def _():
            def slab_base(b):
                return lax.select(b < NBLK, s * E_CH + b * B_E,
                                  (NS + s) * E_CH + (b - NBLK) * B_E)

            def stage_idx(b, slot, sync):
                sbb = slab_base(b)
                for arr, dst in zip((tail2, head2, type2), ivs[slot]):
                    if sync:
                        pltpu.sync_copy(arr.at[pl.ds(sbb, B_E)], dst)
                    else:
                        pltpu.async_copy(arr.at[pl.ds(sbb, B_E)], dst, isem)

            def wait_idx(slot):
                for arr, dst in zip((tail2, head2, type2), ivs[slot]):
                    pltpu.make_async_copy(arr.at[pl.ds(0, B_E)], dst, isem).wait()

            def fire_g(iv3, j, sl):
                pltpu.async_copy(tab.at[iv3[0].at[j]], gb[sl], gsem[sl])
                pltpu.async_copy(w_sp.at[iv3[2].at[j]], wb[sl], wsem[sl])

            def wait_g(sl):
                pltpu.make_async_copy(tab.at[tail_v0.at[0]], gb[sl], gsem[sl]).wait()
                pltpu.make_async_copy(w_sp.at[type_v0.at[0]], wb[sl], wsem[sl]).wait()

            def fire_s(hv, j, sl):
                pltpu.async_copy(sb[sl], acc.at[hv.at[j]], ssem[sl], add=True)

            def wait_s(sl):
                pltpu.make_async_copy(sb[sl], acc.at[head_v0.at[0]], ssem[sl]).wait()

            stage_idx(0, 0, True)
            for j in range(2):
                fire_g(ivs[0], j, j)

            def block_body(p, sig):
                b = 2 * p + sig
                iv3 = ivs[sig]
                hv = iv3[1]
                nv3 = ivs[1 - sig]
                for j in range(B_E):
                    sl = j % 2
                    wait_g(sl)
                    if j >= 2 or sig == 1:
                        wait_s(sl)
                    else:
                        @pl.when(p > 0)
                        def _():
                            wait_s(sl)
                    if j == 2:
                        if sig == 0:
                            stage_idx(b + 1, 1, False)
                        else:
                            @pl.when(p < NBLK - 1)
                            def _():
                                stage_idx(b + 1, 0, False)
                    @pl.loop(0, CHUNK, unroll=8)
                    def _(r):
                        sb[sl][r, pl.ds(0, L)] = gb[sl][r, pl.ds(0, L)] * wb[sl][r, pl.ds(0, L)]
                        sb[sl][r, pl.ds(L, L)] = gb[sl][r, pl.ds(L, L)] * wb[sl][r, pl.ds(L, L)]
                    fire_s(hv, j, sl)
                    if j + 2 < B_E:
                        fire_g(iv3, j + 2, sl)
                    else:
                        if j == B_E - 2:
                            if sig == 0:
                                wait_idx(1)
                            else:
                                @pl.when(p < NBLK - 1)
                                def _():
                                    wait_idx(0)
                        if sig == 0:
                            fire_g(nv3, j + 2 - B_E, sl)
                        else:
                            @pl.when(p < NBLK - 1)
                            def _():
                                fire_g(nv3, j + 2 - B_E, sl)

            @pl.loop(0, NBLK)
            def _(p):
                block_body(p, 0)
                block_body(p, 1)

            wait_s(0)
            wait_s(1)
    plsc.subcore_barrier()

    for core in range(NC):
        outp = (out_lo, out_hi)[core]
        @pl.when(c == core)
        def _():
            @pl.loop(0, 27)
            def _(k):
                pltpu.sync_copy(acc.at[pl.ds(rbase + k * 116, 116)], sb0.at[pl.ds(0, 116)])
                pltpu.sync_copy(sb0.at[pl.ds(0, 116)], outp.at[pl.ds(rbase + k * 116, 116)])


@functools.partial(
    pl.kernel,
    out_type=(jax.ShapeDtypeStruct((ITEM_ACC, HALF), jnp.float32),
              jax.ShapeDtypeStruct((ITEM_ACC, HALF), jnp.float32)),
    mesh=_mesh,
    compiler_params=_sc_params,
    scratch_types=(
        pltpu.VMEM_SHARED((ITEM_ACC, HALF), jnp.float32),
        pltpu.VMEM((B_I, CHUNK), jnp.int32),
        pltpu.VMEM((B_I, CHUNK), jnp.int32),
        pltpu.VMEM((B_I, CHUNK), jnp.int32),
        pltpu.VMEM((B_I, CHUNK), jnp.int32),
        pltpu.VMEM((CHUNK, HALF), jnp.float32),
        pltpu.VMEM((CHUNK, HALF), jnp.float32),
        pltpu.VMEM((CHUNK, HALF), jnp.float32),
        pltpu.VMEM((CHUNK, HALF), jnp.float32),
        pltpu.VMEM((1, HALF), jnp.float32),
        pltpu.SemaphoreType.DMA, pltpu.SemaphoreType.DMA,
        pltpu.SemaphoreType.DMA, pltpu.SemaphoreType.DMA,
        pltpu.SemaphoreType.DMA,
    ),
)
def _sc_iu_agg(u_lo, u_hi, w0_lo, w0_hi, rowg2, cols2,
               out_lo, out_hi,
               acc, row_v0, col_v0, row_v1, col_v1,
               g0, g1, sb0, sb1, wrow,
               sg0, sg1, ss0, ss1, isem):
    c = lax.axis_index("c")
    s = lax.axis_index("s")
    gb = (g0, g1)
    sb = (sb0, sb1)
    gsem = (sg0, sg1)
    ssem = (ss0, ss1)
    ivs = ((row_v0, col_v0), (row_v1, col_v1))
    NBLK = I_CH // B_I
    NBLK2 = 2 * NBLK

    _zero_rows(sb0, CHUNK)
    rbase = s * (ITEM_ACC // NS)
    @pl.loop(0, ITEM_ACC // NS // CHUNK)
    def _(k):
        pltpu.sync_copy(sb0.at[pl.ds(0, CHUNK)], acc.at[pl.ds(rbase + k * CHUNK, CHUNK)])
    plsc.subcore_barrier()

    for core in range(NC):
        tab = (u_lo, u_hi)[core]
        w0t = (w0_lo, w0_hi)[core]
        @pl.when(c == core)
        def _():
            pltpu.sync_copy(w0t, wrow)
            wa = wrow[0, pl.ds(0, L)]
            wvb = wrow[0, pl.ds(L, L)]

            def slab_base(b):
                return lax.select(b < NBLK, s * I_CH + b * B_I,
                                  (NS + s) * I_CH + (b - NBLK) * B_I)

            def stage_idx(b, slot, sync):
                sbb = slab_base(b)
                for arr, dst in zip((rowg2, cols2), ivs[slot]):
                    if sync:
                        pltpu.sync_copy(arr.at[pl.ds(sbb, B_I)], dst)
                    else:
                        pltpu.async_copy(arr.at[pl.ds(sbb, B_I)], dst, isem)

            def wait_idx(slot):
                for arr, dst in zip((rowg2, cols2), ivs[slot]):
                    pltpu.make_async_copy(arr.at[pl.ds(0, B_I)], dst, isem).wait()

            def fire_g(tv, j, sl):
                pltpu.async_copy(tab.at[tv.at[j]], gb[sl], gsem[sl])

            def wait_g(sl):
                pltpu.make_async_copy(tab.at[row_v0.at[0]], gb[sl], gsem[sl]).wait()

            def fire_s(hv, j, sl):
                pltpu.async_copy(sb[sl], acc.at[hv.at[j]], ssem[sl], add=True)

            def wait_s(sl):
                pltpu.make_async_copy(sb[sl], acc.at[col_v0.at[0]], ssem[sl]).wait()

            stage_idx(0, 0, True)
            for j in range(2):
                fire_g(row_v0, j, j)

            def block_body(p, sig):
                b = 2 * p + sig
                tv, hv = ivs[sig]
                nv = ivs[1 - sig][0]
                for j in range(B_I):
                    sl = j % 2
                    wait_g(sl)
                    if j >= 2 or sig == 1:
                        wait_s(sl)
                    else:
                        @pl.when(p > 0)
                        def _():
                            wait_s(sl)
                    if j == 2:
                        if sig == 0:
                            stage_idx(b + 1, 1, False)
                        else:
                            @pl.when(p < NBLK - 1)
                            def _():
                                stage_idx(b + 1, 0, False)
                    @pl.loop(0, CHUNK, unroll=8)
                    def _(r):
                        sb[sl][r, pl.ds(0, L)] = gb[sl][r, pl.ds(0, L)] * wa
                        sb[sl][r, pl.ds(L, L)] = gb[sl][r, pl.ds(L, L)] * wvb
                    fire_s(hv, j, sl)
                    if j + 2 < B_I:
                        fire_g(tv, j + 2, sl)
                    else:
                        if j == B_I - 2:
                            if sig == 0:
                                wait_idx(1)
                            else:
                                @pl.when(p < NBLK - 1)
                                def _():
                                    wait_idx(0)
                        if sig == 0:
                            fire_g(nv, j + 2 - B_I, sl)
                        else:
                            @pl.when(p < NBLK - 1)
                            def _():
                                fire_g(nv, j + 2 - B_I, sl)

            @pl.loop(0, NBLK)
            def _(p):
                block_body(p, 0)
                block_body(p, 1)

            wait_s(0)
            wait_s(1)
    plsc.subcore_barrier()

    for core in range(NC):
        outp = (out_lo, out_hi)[core]
        @pl.when(c == core)
        def _():
            @pl.loop(0, ITEM_ACC // NS // CHUNK)
            def _(k):
                pltpu.sync_copy(acc.at[pl.ds(rbase + k * CHUNK, CHUNK)], sb0.at[pl.ds(0, CHUNK)])
                pltpu.sync_copy(sb0.at[pl.ds(0, CHUNK)], outp.at[pl.ds(rbase + k * CHUNK, CHUNK)])


@functools.partial(
    pl.kernel,
    out_type=(jax.ShapeDtypeStruct((USER_ACC, HALF), jnp.float32),
              jax.ShapeDtypeStruct((USER_ACC, HALF), jnp.float32)),
    mesh=_mesh,
    compiler_params=_sc_params,
    scratch_types=(
        pltpu.VMEM_SHARED((USER_ACC, HALF), jnp.float32),
        pltpu.VMEM((B_I, CHUNK), jnp.int32),
        pltpu.VMEM((B_I, CHUNK), jnp.int32),
        pltpu.VMEM((B_I, CHUNK), jnp.int32),
        pltpu.VMEM((B_I, CHUNK), jnp.int32),
        pltpu.VMEM((CHUNK, HALF), jnp.float32),
        pltpu.VMEM((CHUNK, HALF), jnp.float32),
        pltpu.VMEM((CHUNK, HALF), jnp.float32),
        pltpu.VMEM((CHUNK, HALF), jnp.float32),
        pltpu.SemaphoreType.DMA, pltpu.SemaphoreType.DMA,
        pltpu.SemaphoreType.DMA, pltpu.SemaphoreType.DMA,
        pltpu.SemaphoreType.DMA, pltpu.SemaphoreType.DMA,
        pltpu.SemaphoreType.DMA, pltpu.SemaphoreType.DMA,
        pltpu.SemaphoreType.DMA,
    ),
)
def _sc_user_agg(f_lo, f_hi, colg2, rows2,
                 out_lo, out_hi,
                 acc, col_v0, row_v0, col_v1, row_v1,
                 g0, g1, g2, g3,
                 sg0, sg1, sg2, sg3, ss0, ss1, ss2, ss3, isem):
    c = lax.axis_index("c")
    s = lax.axis_index("s")
    gb = (g0, g1, g2, g3)
    gsem = (sg0, sg1, sg2, sg3)
    ssem = (ss0, ss1, ss2, ss3)
    ivs = ((col_v0, row_v0), (col_v1, row_v1))
    NBLK = I_CH // B_I
    NBLK2 = 2 * NBLK

    _zero_rows(g0, CHUNK)
    rbase = s * (USER_ACC // NS)
    @pl.loop(0, USER_ACC // NS // CHUNK)
    def _(k):
        pltpu.sync_copy(g0.at[pl.ds(0, CHUNK)], acc.at[pl.ds(rbase + k * CHUNK, CHUNK)])
    plsc.subcore_barrier()

    for core in range(NC):
        tab = (f_lo, f_hi)[core]
        @pl.when(c == core)
        def _():
            def slab_base(b):
                return lax.select(b < NBLK, s * I_CH + b * B_I,
                                  (NS + s) * I_CH + (b - NBLK) * B_I)

            def stage_idx(b, slot, sync):
                sbb = slab_base(b)
                for arr, dst in zip((colg2, rows2), ivs[slot]):
                    if sync:
                        pltpu.sync_copy(arr.at[pl.ds(sbb, B_I)], dst)
                    else:
                        pltpu.async_copy(arr.at[pl.ds(sbb, B_I)], dst, isem)

            def wait_idx(slot):
                for arr, dst in zip((colg2, rows2), ivs[slot]):
                    pltpu.make_async_copy(arr.at[pl.ds(0, B_I)], dst, isem).wait()

            def fire_g(tv, j, sl):
                pltpu.async_copy(tab.at[tv.at[j]], gb[sl], gsem[sl])

            def wait_g(sl):
                pltpu.make_async_copy(tab.at[col_v0.at[0]], gb[sl], gsem[sl]).wait()

            def fire_s(hv, j, sl):
                pltpu.async_copy(gb[sl], acc.at[hv.at[j]], ssem[sl], add=True)

            def wait_s(sl):
                pltpu.make_async_copy(gb[sl], acc.at[row_v0.at[0]], ssem[sl]).wait()

            stage_idx(0, 0, True)
            for j in range(2):
                fire_g(col_v0, j, j)

            def block_body(p, sig):
                b = 2 * p + sig
                tv, hv = ivs[sig]
                nv = ivs[1 - sig][0]
                for j in range(B_I):
                    sl = j % 4
                    wait_g(sl)
                    fire_s(hv, j, sl)
                    if j == 2:
                        if sig == 0:
                            stage_idx(b + 1, 1, False)
                        else:
                            @pl.when(p < NBLK - 1)
                            def _():
                                stage_idx(b + 1, 0, False)
                    tsl = (j + 2) % 4
                    if j + 2 < B_I:
                        if j >= 2 or sig == 1:
                            wait_s(tsl)
                            fire_g(tv, j + 2, sl=tsl)
                        else:
                            @pl.when(p > 0)
                            def _():
                                wait_s(tsl)
                            fire_g(tv, j + 2, sl=tsl)
                    else:
                        if j == B_I - 2:
                            if sig == 0:
                                wait_idx(1)
                            else:
                                @pl.when(p < NBLK - 1)
                                def _():
                                    wait_idx(0)
                        if sig == 0:
                            wait_s(tsl)
                            fire_g(nv, j + 2 - B_I, sl=tsl)
                        else:
                            @pl.when(p < NBLK - 1)
                            def _():
                                wait_s(tsl)
                                fire_g(nv, j + 2 - B_I, sl=tsl)

            @pl.loop(0, NBLK)
            def _(p):
                block_body(p, 0)
                block_body(p, 1)

            wait_s(0)
            wait_s(1)
            wait_s(2)
            wait_s(3)
    plsc.subcore_barrier()

    for core in range(NC):
        outp = (out_lo, out_hi)[core]
        @pl.when(c == core)
        def _():
            @pl.loop(0, USER_ACC // NS // CHUNK)
            def _(k):
                pltpu.sync_copy(acc.at[pl.ds(rbase + k * CHUNK, CHUNK)], g0.at[pl.ds(0, CHUNK)])
                pltpu.sync_copy(g0.at[pl.ds(0, CHUNK)], outp.at[pl.ds(rbase + k * CHUNK, CHUNK)])


def _tc_gate(agg_lo, agg_hi, cnt_e, iu_lo, iu_hi, cnt_i, g1t, g2t, res_prev):
    blk = 1000

    def body(alo, ahi, ce, ilo, ihi, ci, g1, g2, rp,
             flo, fhi, elo, ehi, rout):
        ikg = jnp.concatenate([alo[...], ahi[...]], axis=1) / jnp.maximum(ce[...], 1.0)
        iu = jnp.concatenate([ilo[...], ihi[...]], axis=1) / jnp.maximum(ci[...], 1.0)
        z = (jnp.dot(ikg, g1[...], preferred_element_type=jnp.float32)
             + jnp.dot(iu, g2[...], preferred_element_type=jnp.float32))
        gi = jax.nn.sigmoid(z)
        f = gi * ikg + (1.0 - gi) * iu
        flo[...] = f[:, :HALF]
        fhi[...] = f[:, HALF:]
        n = jnp.sqrt(jnp.sum(f * f, axis=1, keepdims=True))
        fn = f / jnp.maximum(n, 1e-12)
        elo[...] = fn[:, :HALF]
        ehi[...] = fn[:, HALF:]
        rout[...] = rp[...] + fn

    half_spec = pl.BlockSpec((blk, HALF), lambda i: (i, 0))
    cnt_spec = pl.BlockSpec((blk, 1), lambda i: (i, 0))
    mat_spec = pl.BlockSpec((DIM, DIM), lambda i: (0, 0))
    full_spec = pl.BlockSpec((blk, DIM), lambda i: (i, 0))
    return pl.pallas_call(
        body,
        grid=(N_ITEMS // blk,),
        in_specs=[half_spec, half_spec, cnt_spec, half_spec, half_spec,
                  cnt_spec, mat_spec, mat_spec, full_spec],
        out_specs=[half_spec, half_spec, half_spec, half_spec, full_spec],
        out_shape=[
            jax.ShapeDtypeStruct((N_ITEMS, HALF), jnp.float32),
            jax.ShapeDtypeStruct((N_ITEMS, HALF), jnp.float32),
            jax.ShapeDtypeStruct((N_ITEMS, HALF), jnp.float32),
            jax.ShapeDtypeStruct((N_ITEMS, HALF), jnp.float32),
            jax.ShapeDtypeStruct((N_ITEMS, DIM), jnp.float32),
        ],
    )(agg_lo, agg_hi, cnt_e, iu_lo, iu_hi, cnt_i, g1t, g2t, res_prev)


def _tc_normres(x_lo, x_hi, res_prev):
    n_rows = x_lo.shape[0]
    blk = 1000

    def body(xlo, xhi, rp, nlo, nhi, rout):
        x = jnp.concatenate([xlo[...], xhi[...]], axis=1)
        n = jnp.sqrt(jnp.sum(x * x, axis=1, keepdims=True))
        xn = x / jnp.maximum(n, 1e-12)
        nlo[...] = xn[:, :HALF]
        nhi[...] = xn[:, HALF:]
        rout[...] = rp[...] + xn

    half_spec = pl.BlockSpec((blk, HALF), lambda i: (i, 0))
    full_spec = pl.BlockSpec((blk, DIM), lambda i: (i, 0))
    return pl.pallas_call(
        body,
        grid=(n_rows // blk,),
        in_specs=[half_spec, half_spec, full_spec],
        out_specs=[half_spec, half_spec, full_spec],
        out_shape=[
            jax.ShapeDtypeStruct((n_rows, HALF), jnp.float32),
            jax.ShapeDtypeStruct((n_rows, HALF), jnp.float32),
            jax.ShapeDtypeStruct((n_rows, DIM), jnp.float32),
        ],
    )(x_lo, x_hi, res_prev)


def _pack(x, nch, padval):
    tot = NW * nch * CHUNK
    return jnp.pad(x.astype(jnp.int32), (0, tot - x.shape[0]),
                   constant_values=padval).reshape(NW * nch, CHUNK)


def kernel(user_emb, entity_emb, edge_index, edge_type, mat_row, mat_col, mat_val,
           weight, gate1_w0, gate2_w0, gate1_w1, gate2_w1):
    head = edge_index[0]
    tail = edge_index[1]
    tail2 = _pack(tail, E_CH, 0)
    head2 = _pack(head, E_CH, N_ENTITIES)
    type2 = _pack(edge_type, E_CH, 0)
    rowg2 = _pack(mat_row, I_CH, 0)
    rows2 = _pack(mat_row, I_CH, N_USERS)
    colg2 = _pack(mat_col, I_CH, 0)
    cols2 = _pack(mat_col, I_CH, N_ITEMS)

    cnt_e_raw, cnt_i_raw = _sc_counts(head2, cols2)
    cnt_e = cnt_e_raw[:N_ITEMS].reshape(N_ITEMS, 1)
    cnt_i = cnt_i_raw[:N_ITEMS].reshape(N_ITEMS, 1)

    e_lo, e_hi = entity_emb[:, :HALF], entity_emb[:, HALF:]
    u_lo, u_hi = user_emb[:, :HALF], user_emb[:, HALF:]
    w_lo, w_hi = weight[:, :HALF], weight[:, HALF:]
    w0_lo, w0_hi = weight[0:1, :HALF], weight[0:1, HALF:]
    g1t = (gate1_w0.T, gate1_w1.T)
    g2t = (gate2_w0.T, gate2_w1.T)

    res_i = entity_emb[:N_ITEMS]
    res_a = entity_emb[N_ITEMS:]
    res_u = user_emb

    for i in range(N_HOPS):
        agg_lo, agg_hi = _sc_kg_agg(e_lo, e_hi, w_lo, w_hi, tail2, head2, type2)
        iu_lo, iu_hi = _sc_iu_agg(u_lo, u_hi, w0_lo, w0_hi, rowg2, cols2)
        f_lo, f_hi, en_lo, en_hi, res_i = _tc_gate(
            agg_lo[:N_ITEMS], agg_hi[:N_ITEMS], cnt_e,
            iu_lo[:N_ITEMS], iu_hi[:N_ITEMS], cnt_i, g1t[i], g2t[i], res_i)
        us_lo, us_hi = _sc_user_agg(f_lo, f_hi, colg2, rows2)
        an_lo, an_hi, res_a = _tc_normres(
            agg_lo[N_ITEMS:N_ENTITIES], agg_hi[N_ITEMS:N_ENTITIES], res_a)
        un_lo, un_hi, res_u = _tc_normres(us_lo[:N_USERS], us_hi[:N_USERS], res_u)
        if i + 1 < N_HOPS:
            e_lo = jnp.concatenate([en_lo, an_lo], axis=0)
            e_hi = jnp.concatenate([en_hi, an_hi], axis=0)
            u_lo, u_hi = un_lo, un_hi

    entity_res = jnp.concatenate([res_i, res_a], axis=0)
    return (entity_res, res_u)


# speedup vs baseline: 4.5050x; 1.0012x over previous
import functools

import jax
import jax.numpy as jnp
from jax import lax
from jax.experimental import pallas as pl
from jax.experimental.pallas import tpu as pltpu
from jax.experimental.pallas import tpu_sc as plsc

N_USERS = 30000
N_ITEMS = 20000
N_ENTITIES = 50000
N_RELATIONS = 16
DIM = 64
HALF = 32
N_HOPS = 2
N_EDGES = 800000
N_INTER = 500000

NC = 2
NS = 16
NW = NC * NS
L = 16
CHUNK = 128

E_CH = 200
I_CH = 128
B_E = 8
B_I = 8

ENT_ACC = 50112
ENT_CNT = 50176
ITEM_ACC = 20480
USER_ACC = 30720

_mesh = plsc.VectorSubcoreMesh(
    core_axis_name="c", subcore_axis_name="s", num_cores=NC, num_subcores=NS)
_sc_params = pltpu.CompilerParams(use_tc_tiling_on_sc=False)


def _zero_rows(buf, nrows):
    @pl.loop(0, nrows, unroll=8)
    def _(r):
        z = jnp.zeros((L,), jnp.float32)
        buf[r, pl.ds(0, L)] = z
        buf[r, pl.ds(L, L)] = z


def _zero_flat(buf, n):
    @pl.loop(0, n // L, unroll=8)
    def _(k):
        buf[pl.ds(k * L, L)] = jnp.zeros((L,), jnp.float32)


@functools.partial(
    pl.kernel,
    out_type=(jax.ShapeDtypeStruct((ENT_CNT,), jnp.float32),
              jax.ShapeDtypeStruct((ITEM_ACC,), jnp.float32)),
    mesh=_mesh,
    compiler_params=_sc_params,
    scratch_types=(
        pltpu.VMEM_SHARED((ENT_CNT,), jnp.float32),
        pltpu.VMEM((B_E, CHUNK), jnp.int32),
        pltpu.VMEM((CHUNK,), jnp.float32),
        pltpu.VMEM((3136,), jnp.float32),
    ),
)
def _sc_counts(head2, cols2, cnt_e, cnt_i, acc, idx_v, ones_v, stage_v):
    c = lax.axis_index("c")
    s = lax.axis_index("s")
    @pl.loop(0, CHUNK // L, unroll=8)
    def _(k):
        ones_v[pl.ds(k * L, L)] = jnp.ones((L,), jnp.float32)
    _zero_flat(stage_v, 3136)

    @pl.when(c == 0)
    def _():
        pltpu.sync_copy(stage_v, acc.at[pl.ds(s * 3136, 3136)])
    @pl.when(c == 1)
    def _():
        pltpu.sync_copy(stage_v.at[pl.ds(0, 1280)], acc.at[pl.ds(s * 1280, 1280)])
    plsc.subcore_barrier()

    @pl.when(c == 0)
    def _():
        for half in range(2):
            base = (half * NS + s) * E_CH
            @pl.loop(0, E_CH // B_E)
            def _(bj):
                pltpu.sync_copy(head2.at[pl.ds(base + bj * B_E, B_E)], idx_v)
                @pl.loop(0, B_E)
                def _(j):
                    pltpu.sync_copy(ones_v, acc.at[idx_v.at[j]], add=True)
    @pl.when(c == 1)
    def _():
        for half in range(2):
            base = (half * NS + s) * I_CH
            @pl.loop(0, I_CH // B_E)
            def _(bj):
                pltpu.sync_copy(cols2.at[pl.ds(base + bj * B_E, B_E)], idx_v)
                @pl.loop(0, B_E)
                def _(j):
                    pltpu.sync_copy(ones_v, acc.at[idx_v.at[j]], add=True)
    plsc.subcore_barrier()

    @pl.when(c == 0)
    def _():
        pltpu.sync_copy(acc.at[pl.ds(s * 3136, 3136)], stage_v)
        pltpu.sync_copy(stage_v, cnt_e.at[pl.ds(s * 3136, 3136)])
    @pl.when(c == 1)
    def _():
        pltpu.sync_copy(acc.at[pl.ds(s * 1280, 1280)], stage_v.at[pl.ds(0, 1280)])
        pltpu.sync_copy(stage_v.at[pl.ds(0, 1280)], cnt_i.at[pl.ds(s * 1280, 1280)])


@functools.partial(
    pl.kernel,
    out_type=(jax.ShapeDtypeStruct((ENT_ACC, HALF), jnp.float32),
              jax.ShapeDtypeStruct((ENT_ACC, HALF), jnp.float32)),
    mesh=_mesh,
    compiler_params=_sc_params,
    scratch_types=(
        pltpu.VMEM_SHARED((ENT_ACC, HALF), jnp.float32),
        pltpu.VMEM_SHARED((N_RELATIONS, HALF), jnp.float32),
        pltpu.VMEM((B_E, CHUNK), jnp.int32),
        pltpu.VMEM((B_E, CHUNK), jnp.int32),
        pltpu.VMEM((B_E, CHUNK), jnp.int32),
        pltpu.VMEM((B_E, CHUNK), jnp.int32),
        pltpu.VMEM((B_E, CHUNK), jnp.int32),
        pltpu.VMEM((B_E, CHUNK), jnp.int32),
        pltpu.VMEM((CHUNK, HALF), jnp.float32),
        pltpu.VMEM((CHUNK, HALF), jnp.float32),
        pltpu.VMEM((CHUNK, HALF), jnp.float32),
        pltpu.VMEM((CHUNK, HALF), jnp.float32),
        pltpu.VMEM((CHUNK, HALF), jnp.float32),
        pltpu.VMEM((CHUNK, HALF), jnp.float32),
        pltpu.SemaphoreType.DMA, pltpu.SemaphoreType.DMA,
        pltpu.SemaphoreType.DMA, pltpu.SemaphoreType.DMA,
        pltpu.SemaphoreType.DMA, pltpu.SemaphoreType.DMA,
        pltpu.SemaphoreType.DMA,
    ),
)
def _sc_kg_agg(e_lo, e_hi, w_lo, w_hi, tail2, head2, type2,
               out_lo, out_hi,
               acc, w_sp, tail_v0, head_v0, type_v0, tail_v1, head_v1, type_v1,
               g0, g1, w0b, w1b, sb0, sb1,
               sg0, sg1, sw0, sw1, ss0, ss1, isem):
    c = lax.axis_index("c")
    s = lax.axis_index("s")
    gb = (g0, g1)
    wb = (w0b, w1b)
    sb = (sb0, sb1)
    gsem = (sg0, sg1)
    wsem = (sw0, sw1)
    ssem = (ss0, ss1)
    ivs = ((tail_v0, head_v0, type_v0), (tail_v1, head_v1, type_v1))
    NBLK = E_CH // B_E
    NBLK2 = 2 * NBLK

    _zero_rows(sb0, CHUNK)
    rbase = s * (ENT_ACC // NS)
    @pl.loop(0, 27)
    def _(k):
        pltpu.sync_copy(sb0.at[pl.ds(0, 116)], acc.at[pl.ds(rbase + k * 116, 116)])
    for core in range(NC):
        @pl.when((c == core) & (s == 0))
        def _():
            pltpu.sync_copy((w_lo, w_hi)[core], sb1.at[pl.ds(0, N_RELATIONS)])
            pltpu.sync_copy(sb1.at[pl.ds(0, N_RELATIONS)], w_sp)
    plsc.subcore_barrier()

    for core in range(NC):
        tab = (e_lo, e_hi)[core]
        @pl.when(c == core)
        def _():
            def slab_base(b):
                return lax.select(b < NBLK, s * E_CH + b * B_E,
                                  (NS + s) * E_CH + (b - NBLK) * B_E)

            def stage_idx(b, slot, sync):
                sbb = slab_base(b)
                for arr, dst in zip((tail2, head2, type2), ivs[slot]):
                    if sync:
                        pltpu.sync_copy(arr.at[pl.ds(sbb, B_E)], dst)
                    else:
                        pltpu.async_copy(arr.at[pl.ds(sbb, B_E)], dst, isem)

            def wait_idx(slot):
                for arr, dst in zip((tail2, head2, type2), ivs[slot]):
                    pltpu.make_async_copy(arr.at[pl.ds(0, B_E)], dst, isem).wait()

            def fire_g(iv3, j, sl):
                pltpu.async_copy(tab.at[iv3[0].at[j]], gb[sl], gsem[sl])
                pltpu.async_copy(w_sp.at[iv3[2].at[j]], wb[sl], wsem[sl])

            def wait_g(sl):
                pltpu.make_async_copy(tab.at[tail_v0.at[0]], gb[sl], gsem[sl]).wait()
                pltpu.make_async_copy(w_sp.at[type_v0.at[0]], wb[sl], wsem[sl]).wait()

            def fire_s(hv, j, sl):
                pltpu.async_copy(sb[sl], acc.at[hv.at[j]], ssem[sl], add=True)

            def wait_s(sl):
                pltpu.make_async_copy(sb[sl], acc.at[head_v0.at[0]], ssem[sl]).wait()

            stage_idx(0, 0, True)
            for j in range(2):
                fire_g(ivs[0], j, j)

            def block_body(p, sig):
                b = 2 * p + sig
                iv3 = ivs[sig]
                hv = iv3[1]
                nv3 = ivs[1 - sig]
                for j in range(B_E):
                    sl = j % 2
                    wait_g(sl)
                    if j >= 2 or sig == 1:
                        wait_s(sl)
                    else:
                        @pl.when(p > 0)
                        def _():
                            wait_s(sl)
                    if j == 2:
                        if sig == 0:
                            stage_idx(b + 1, 1, False)
                        else:
                            @pl.when(p < NBLK - 1)
                            def _():
                                stage_idx(b + 1, 0, False)
                    @pl.loop(0, CHUNK, unroll=8)
                    def _(r):
                        sb[sl][r, pl.ds(0, L)] = gb[sl][r, pl.ds(0, L)] * wb[sl][r, pl.ds(0, L)]
                        sb[sl][r, pl.ds(L, L)] = gb[sl][r, pl.ds(L, L)] * wb[sl][r, pl.ds(L, L)]
                    fire_s(hv, j, sl)
                    if j + 2 < B_E:
                        fire_g(iv3, j + 2, sl)
                    else:
                        if j == B_E - 2:
                            if sig == 0:
                                wait_idx(1)
                            else:
                                @pl.when(p < NBLK - 1)
                                def _():
                                    wait_idx(0)
                        if sig == 0:
                            fire_g(nv3, j + 2 - B_E, sl)
                        else:
                            @pl.when(p < NBLK - 1)
                            def _():
                                fire_g(nv3, j + 2 - B_E, sl)

            @pl.loop(0, NBLK)
            def _(p):
                block_body(p, 0)
                block_body(p, 1)

            wait_s(0)
            wait_s(1)
    plsc.subcore_barrier()

    for core in range(NC):
        outp = (out_lo, out_hi)[core]
        @pl.when(c == core)
        def _():
            @pl.loop(0, 27)
            def _(k):
                pltpu.sync_copy(acc.at[pl.ds(rbase + k * 116, 116)], sb0.at[pl.ds(0, 116)])
                pltpu.sync_copy(sb0.at[pl.ds(0, 116)], outp.at[pl.ds(rbase + k * 116, 116)])


@functools.partial(
    pl.kernel,
    out_type=(jax.ShapeDtypeStruct((ITEM_ACC, HALF), jnp.float32),
              jax.ShapeDtypeStruct((ITEM_ACC, HALF), jnp.float32)),
    mesh=_mesh,
    compiler_params=_sc_params,
    scratch_types=(
        pltpu.VMEM_SHARED((ITEM_ACC, HALF), jnp.float32),
        pltpu.VMEM((B_I, CHUNK), jnp.int32),
        pltpu.VMEM((B_I, CHUNK), jnp.int32),
        pltpu.VMEM((B_I, CHUNK), jnp.int32),
        pltpu.VMEM((B_I, CHUNK), jnp.int32),
        pltpu.VMEM((CHUNK, HALF), jnp.float32),
        pltpu.VMEM((CHUNK, HALF), jnp.float32),
        pltpu.VMEM((CHUNK, HALF), jnp.float32),
        pltpu.VMEM((CHUNK, HALF), jnp.float32),
        pltpu.VMEM((1, HALF), jnp.float32),
        pltpu.SemaphoreType.DMA, pltpu.SemaphoreType.DMA,
        pltpu.SemaphoreType.DMA, pltpu.SemaphoreType.DMA,
        pltpu.SemaphoreType.DMA,
    ),
)
def _sc_iu_agg(u_lo, u_hi, w0_lo, w0_hi, rowg2, cols2,
               out_lo, out_hi,
               acc, row_v0, col_v0, row_v1, col_v1,
               g0, g1, sb0, sb1, wrow,
               sg0, sg1, ss0, ss1, isem):
    c = lax.axis_index("c")
    s = lax.axis_index("s")
    gb = (g0, g1)
    sb = (sb0, sb1)
    gsem = (sg0, sg1)
    ssem = (ss0, ss1)
    ivs = ((row_v0, col_v0), (row_v1, col_v1))
    NBLK = I_CH // B_I
    NBLK2 = 2 * NBLK

    _zero_rows(sb0, CHUNK)
    rbase = s * (ITEM_ACC // NS)
    @pl.loop(0, ITEM_ACC // NS // CHUNK)
    def _(k):
        pltpu.sync_copy(sb0.at[pl.ds(0, CHUNK)], acc.at[pl.ds(rbase + k * CHUNK, CHUNK)])
    plsc.subcore_barrier()

    for core in range(NC):
        tab = (u_lo, u_hi)[core]
        w0t = (w0_lo, w0_hi)[core]
        @pl.when(c == core)
        def _():
            pltpu.sync_copy(w0t, wrow)
            wa = wrow[0, pl.ds(0, L)]
            wvb = wrow[0, pl.ds(L, L)]

            def slab_base(b):
                return lax.select(b < NBLK, s * I_CH + b * B_I,
                                  (NS + s) * I_CH + (b - NBLK) * B_I)

            def stage_idx(b, slot, sync):
                sbb = slab_base(b)
                for arr, dst in zip((rowg2, cols2), ivs[slot]):
                    if sync:
                        pltpu.sync_copy(arr.at[pl.ds(sbb, B_I)], dst)
                    else:
                        pltpu.async_copy(arr.at[pl.ds(sbb, B_I)], dst, isem)

            def wait_idx(slot):
                for arr, dst in zip((rowg2, cols2), ivs[slot]):
                    pltpu.make_async_copy(arr.at[pl.ds(0, B_I)], dst, isem).wait()

            def fire_g(tv, j, sl):
                pltpu.async_copy(tab.at[tv.at[j]], gb[sl], gsem[sl])

            def wait_g(sl):
                pltpu.make_async_copy(tab.at[row_v0.at[0]], gb[sl], gsem[sl]).wait()

            def fire_s(hv, j, sl):
                pltpu.async_copy(sb[sl], acc.at[hv.at[j]], ssem[sl], add=True)

            def wait_s(sl):
                pltpu.make_async_copy(sb[sl], acc.at[col_v0.at[0]], ssem[sl]).wait()

            stage_idx(0, 0, True)
            for j in range(2):
                fire_g(row_v0, j, j)

            def block_body(p, sig):
                b = 2 * p + sig
                tv, hv = ivs[sig]
                nv = ivs[1 - sig][0]
                for j in range(B_I):
                    sl = j % 2
                    wait_g(sl)
                    if j >= 2 or sig == 1:
                        wait_s(sl)
                    else:
                        @pl.when(p > 0)
                        def _():
                            wait_s(sl)
                    if j == 2:
                        if sig == 0:
                            stage_idx(b + 1, 1, False)
                        else:
                            @pl.when(p < NBLK - 1)
                            def _():
                                stage_idx(b + 1, 0, False)
                    @pl.loop(0, CHUNK, unroll=8)
                    def _(r):
                        sb[sl][r, pl.ds(0, L)] = gb[sl][r, pl.ds(0, L)] * wa
                        sb[sl][r, pl.ds(L, L)] = gb[sl][r, pl.ds(L, L)] * wvb
                    fire_s(hv, j, sl)
                    if j + 2 < B_I:
                        fire_g(tv, j + 2, sl)
                    else:
                        if j == B_I - 2:
                            if sig == 0:
                                wait_idx(1)
                            else:
                                @pl.when(p < NBLK - 1)
                                def _():
                                    wait_idx(0)
                        if sig == 0:
                            fire_g(nv, j + 2 - B_I, sl)
                        else:
                            @pl.when(p < NBLK - 1)
                            def _():
                                fire_g(nv, j + 2 - B_I, sl)

            @pl.loop(0, NBLK)
            def _(p):
                block_body(p, 0)
                block_body(p, 1)

            wait_s(0)
            wait_s(1)
    plsc.subcore_barrier()

    for core in range(NC):
        outp = (out_lo, out_hi)[core]
        @pl.when(c == core)
        def _():
            @pl.loop(0, ITEM_ACC // NS // CHUNK)
            def _(k):
                pltpu.sync_copy(acc.at[pl.ds(rbase + k * CHUNK, CHUNK)], sb0.at[pl.ds(0, CHUNK)])
                pltpu.sync_copy(sb0.at[pl.ds(0, CHUNK)], outp.at[pl.ds(rbase + k * CHUNK, CHUNK)])


@functools.partial(
    pl.kernel,
    out_type=(jax.ShapeDtypeStruct((USER_ACC, HALF), jnp.float32),
              jax.ShapeDtypeStruct((USER_ACC, HALF), jnp.float32)),
    mesh=_mesh,
    compiler_params=_sc_params,
    scratch_types=(
        pltpu.VMEM_SHARED((USER_ACC, HALF), jnp.float32),
        pltpu.VMEM((B_I, CHUNK), jnp.int32),
        pltpu.VMEM((B_I, CHUNK), jnp.int32),
        pltpu.VMEM((B_I, CHUNK), jnp.int32),
        pltpu.VMEM((B_I, CHUNK), jnp.int32),
        pltpu.VMEM((CHUNK, HALF), jnp.float32),
        pltpu.VMEM((CHUNK, HALF), jnp.float32),
        pltpu.VMEM((CHUNK, HALF), jnp.float32),
        pltpu.VMEM((CHUNK, HALF), jnp.float32),
        pltpu.VMEM((CHUNK, HALF), jnp.float32),
        pltpu.VMEM((CHUNK, HALF), jnp.float32),
        pltpu.VMEM((CHUNK, HALF), jnp.float32),
        pltpu.VMEM((CHUNK, HALF), jnp.float32),
        pltpu.SemaphoreType.DMA, pltpu.SemaphoreType.DMA,
        pltpu.SemaphoreType.DMA, pltpu.SemaphoreType.DMA,
        pltpu.SemaphoreType.DMA, pltpu.SemaphoreType.DMA,
        pltpu.SemaphoreType.DMA, pltpu.SemaphoreType.DMA,
        pltpu.SemaphoreType.DMA, pltpu.SemaphoreType.DMA,
        pltpu.SemaphoreType.DMA, pltpu.SemaphoreType.DMA,
        pltpu.SemaphoreType.DMA, pltpu.SemaphoreType.DMA,
        pltpu.SemaphoreType.DMA, pltpu.SemaphoreType.DMA,
        pltpu.SemaphoreType.DMA,
    ),
)
def _sc_user_agg(f_lo, f_hi, colg2, rows2,
                 out_lo, out_hi,
                 acc, col_v0, row_v0, col_v1, row_v1,
                 g0, g1, g2, g3, g4, g5, g6, g7,
                 sg0, sg1, sg2, sg3, sg4, sg5, sg6, sg7,
                 ss0, ss1, ss2, ss3, ss4, ss5, ss6, ss7, isem):
    c = lax.axis_index("c")
    s = lax.axis_index("s")
    gb = (g0, g1, g2, g3, g4, g5, g6, g7)
    gsem = (sg0, sg1, sg2, sg3, sg4, sg5, sg6, sg7)
    ssem = (ss0, ss1, ss2, ss3, ss4, ss5, ss6, ss7)
    ivs = ((col_v0, row_v0), (col_v1, row_v1))
    NBLK = I_CH // B_I
    PD = 4

    _zero_rows(g0, CHUNK)
    rbase = s * (USER_ACC // NS)
    @pl.loop(0, USER_ACC // NS // CHUNK)
    def _(k):
        pltpu.sync_copy(g0.at[pl.ds(0, CHUNK)], acc.at[pl.ds(rbase + k * CHUNK, CHUNK)])
    plsc.subcore_barrier()

    for core in range(NC):
        tab = (f_lo, f_hi)[core]
        @pl.when(c == core)
        def _():
            def slab_base(b):
                return lax.select(b < NBLK, s * I_CH + b * B_I,
                                  (NS + s) * I_CH + (b - NBLK) * B_I)

            def stage_idx(b, slot, sync):
                sbb = slab_base(b)
                for arr, dst in zip((colg2, rows2), ivs[slot]):
                    if sync:
                        pltpu.sync_copy(arr.at[pl.ds(sbb, B_I)], dst)
                    else:
                        pltpu.async_copy(arr.at[pl.ds(sbb, B_I)], dst, isem)

            def wait_idx(slot):
                for arr, dst in zip((colg2, rows2), ivs[slot]):
                    pltpu.make_async_copy(arr.at[pl.ds(0, B_I)], dst, isem).wait()

            def fire_g(tv, j, sl):
                pltpu.async_copy(tab.at[tv.at[j]], gb[sl], gsem[sl])

            def wait_g(sl):
                pltpu.make_async_copy(tab.at[col_v0.at[0]], gb[sl], gsem[sl]).wait()

            def fire_s(hv, j, sl):
                pltpu.async_copy(gb[sl], acc.at[hv.at[j]], ssem[sl], add=True)

            def wait_s(sl):
                pltpu.make_async_copy(gb[sl], acc.at[row_v0.at[0]], ssem[sl]).wait()

            stage_idx(0, 0, True)
            for j in range(PD):
                fire_g(col_v0, j, j)

            def block_body(p, sig):
                b = 2 * p + sig
                tv, hv = ivs[sig]
                nv = ivs[1 - sig][0]
                for j in range(B_I):
                    sl = j % 8
                    wait_g(sl)
                    fire_s(hv, j, sl)
                    if j == 2:
                        if sig == 0:
                            stage_idx(b + 1, 1, False)
                        else:
                            @pl.when(p < NBLK - 1)
                            def _():
                                stage_idx(b + 1, 0, False)
                    tsl = (j + PD) % 8
                    if j + PD < B_I:
                        if sig == 1:
                            wait_s(tsl)
                            fire_g(tv, j + PD, tsl)
                        else:
                            @pl.when(p > 0)
                            def _():
                                wait_s(tsl)
                            fire_g(tv, j + PD, tsl)
                    else:
                        if j == B_I - PD:
                            if sig == 0:
                                wait_idx(1)
                            else:
                                @pl.when(p < NBLK - 1)
                                def _():
                                    wait_idx(0)
                        if sig == 0:
                            wait_s(tsl)
                            fire_g(nv, j + PD - B_I, tsl)
                        else:
                            @pl.when(p < NBLK - 1)
                            def _():
                                wait_s(tsl)
                                fire_g(nv, j + PD - B_I, tsl)

            @pl.loop(0, NBLK)
            def _(p):
                block_body(p, 0)
                block_body(p, 1)

            for sl in range(8):
                wait_s(sl)
    plsc.subcore_barrier()

    for core in range(NC):
        outp = (out_lo, out_hi)[core]
        @pl.when(c == core)
        def _():
            @pl.loop(0, USER_ACC // NS // CHUNK)
            def _(k):
                pltpu.sync_copy(acc.at[pl.ds(rbase + k * CHUNK, CHUNK)], g0.at[pl.ds(0, CHUNK)])
                pltpu.sync_copy(g0.at[pl.ds(0, CHUNK)], outp.at[pl.ds(rbase + k * CHUNK, CHUNK)])


def _tc_gate(agg_lo, agg_hi, cnt_e, iu_lo, iu_hi, cnt_i, g1t, g2t, res_prev):
    blk = 1000

    def body(alo, ahi, ce, ilo, ihi, ci, g1, g2, rp,
             flo, fhi, elo, ehi, rout):
        ikg = jnp.concatenate([alo[...], ahi[...]], axis=1) / jnp.maximum(ce[...], 1.0)
        iu = jnp.concatenate([ilo[...], ihi[...]], axis=1) / jnp.maximum(ci[...], 1.0)
        z = (jnp.dot(ikg, g1[...], preferred_element_type=jnp.float32)
             + jnp.dot(iu, g2[...], preferred_element_type=jnp.float32))
        gi = jax.nn.sigmoid(z)
        f = gi * ikg + (1.0 - gi) * iu
        flo[...] = f[:, :HALF]
        fhi[...] = f[:, HALF:]
        n = jnp.sqrt(jnp.sum(f * f, axis=1, keepdims=True))
        fn = f / jnp.maximum(n, 1e-12)
        elo[...] = fn[:, :HALF]
        ehi[...] = fn[:, HALF:]
        rout[...] = rp[...] + fn

    half_spec = pl.BlockSpec((blk, HALF), lambda i: (i, 0))
    cnt_spec = pl.BlockSpec((blk, 1), lambda i: (i, 0))
    mat_spec = pl.BlockSpec((DIM, DIM), lambda i: (0, 0))
    full_spec = pl.BlockSpec((blk, DIM), lambda i: (i, 0))
    return pl.pallas_call(
        body,
        grid=(N_ITEMS // blk,),
        in_specs=[half_spec, half_spec, cnt_spec, half_spec, half_spec,
                  cnt_spec, mat_spec, mat_spec, full_spec],
        out_specs=[half_spec, half_spec, half_spec, half_spec, full_spec],
        out_shape=[
            jax.ShapeDtypeStruct((N_ITEMS, HALF), jnp.float32),
            jax.ShapeDtypeStruct((N_ITEMS, HALF), jnp.float32),
            jax.ShapeDtypeStruct((N_ITEMS, HALF), jnp.float32),
            jax.ShapeDtypeStruct((N_ITEMS, HALF), jnp.float32),
            jax.ShapeDtypeStruct((N_ITEMS, DIM), jnp.float32),
        ],
    )(agg_lo, agg_hi, cnt_e, iu_lo, iu_hi, cnt_i, g1t, g2t, res_prev)


def _tc_normres(x_lo, x_hi, res_prev):
    n_rows = x_lo.shape[0]
    blk = 1000

    def body(xlo, xhi, rp, nlo, nhi, rout):
        x = jnp.concatenate([xlo[...], xhi[...]], axis=1)
        n = jnp.sqrt(jnp.sum(x * x, axis=1, keepdims=True))
        xn = x / jnp.maximum(n, 1e-12)
        nlo[...] = xn[:, :HALF]
        nhi[...] = xn[:, HALF:]
        rout[...] = rp[...] + xn

    half_spec = pl.BlockSpec((blk, HALF), lambda i: (i, 0))
    full_spec = pl.BlockSpec((blk, DIM), lambda i: (i, 0))
    return pl.pallas_call(
        body,
        grid=(n_rows // blk,),
        in_specs=[half_spec, half_spec, full_spec],
        out_specs=[half_spec, half_spec, full_spec],
        out_shape=[
            jax.ShapeDtypeStruct((n_rows, HALF), jnp.float32),
            jax.ShapeDtypeStruct((n_rows, HALF), jnp.float32),
            jax.ShapeDtypeStruct((n_rows, DIM), jnp.float32),
        ],
    )(x_lo, x_hi, res_prev)


def _pack(x, nch, padval):
    tot = NW * nch * CHUNK
    return jnp.pad(x.astype(jnp.int32), (0, tot - x.shape[0]),
                   constant_values=padval).reshape(NW * nch, CHUNK)


def kernel(user_emb, entity_emb, edge_index, edge_type, mat_row, mat_col, mat_val,
           weight, gate1_w0, gate2_w0, gate1_w1, gate2_w1):
    head = edge_index[0]
    tail = edge_index[1]
    tail2 = _pack(tail, E_CH, 0)
    head2 = _pack(head, E_CH, N_ENTITIES)
    type2 = _pack(edge_type, E_CH, 0)
    rowg2 = _pack(mat_row, I_CH, 0)
    rows2 = _pack(mat_row, I_CH, N_USERS)
    colg2 = _pack(mat_col, I_CH, 0)
    cols2 = _pack(mat_col, I_CH, N_ITEMS)

    cnt_e_raw, cnt_i_raw = _sc_counts(head2, cols2)
    cnt_e = cnt_e_raw[:N_ITEMS].reshape(N_ITEMS, 1)
    cnt_i = cnt_i_raw[:N_ITEMS].reshape(N_ITEMS, 1)

    e_lo, e_hi = entity_emb[:, :HALF], entity_emb[:, HALF:]
    u_lo, u_hi = user_emb[:, :HALF], user_emb[:, HALF:]
    w_lo, w_hi = weight[:, :HALF], weight[:, HALF:]
    w0_lo, w0_hi = weight[0:1, :HALF], weight[0:1, HALF:]
    g1t = (gate1_w0.T, gate1_w1.T)
    g2t = (gate2_w0.T, gate2_w1.T)

    res_i = entity_emb[:N_ITEMS]
    res_a = entity_emb[N_ITEMS:]
    res_u = user_emb

    for i in range(N_HOPS):
        agg_lo, agg_hi = _sc_kg_agg(e_lo, e_hi, w_lo, w_hi, tail2, head2, type2)
        iu_lo, iu_hi = _sc_iu_agg(u_lo, u_hi, w0_lo, w0_hi, rowg2, cols2)
        f_lo, f_hi, en_lo, en_hi, res_i = _tc_gate(
            agg_lo[:N_ITEMS], agg_hi[:N_ITEMS], cnt_e,
            iu_lo[:N_ITEMS], iu_hi[:N_ITEMS], cnt_i, g1t[i], g2t[i], res_i)
        us_lo, us_hi = _sc_user_agg(f_lo, f_hi, colg2, rows2)
        an_lo, an_hi, res_a = _tc_normres(
            agg_lo[N_ITEMS:N_ENTITIES], agg_hi[N_ITEMS:N_ENTITIES], res_a)
        un_lo, un_hi, res_u = _tc_normres(us_lo[:N_USERS], us_hi[:N_USERS], res_u)
        if i + 1 < N_HOPS:
            e_lo = jnp.concatenate([en_lo, an_lo], axis=0)
            e_hi = jnp.concatenate([en_hi, an_hi], axis=0)
            u_lo, u_hi = un_lo, un_hi

    entity_res = jnp.concatenate([res_i, res_a], axis=0)
    return (entity_res, res_u)
